# baseline (speedup 1.0000x reference)
import jax
import jax.numpy as jnp
from jax.experimental import pallas as pl
from jax.experimental.pallas import tpu as pltpu

_H = 4
_HID = 24
_G = 512


_BN_BLK = 10000


def _bn_stats_kernel(h_ref, s_ref, q_ref):
    i = pl.program_id(0)

    @pl.when(i == 0)
    def _():
        s_ref[...] = jnp.zeros_like(s_ref)
        q_ref[...] = jnp.zeros_like(q_ref)

    h = h_ref[...]
    s_ref[...] += jnp.sum(h, axis=0, keepdims=True)
    q_ref[...] += jnp.sum(h * h, axis=0, keepdims=True)


def _bn_apply_kernel(h_ref, mu_ref, isd_ref, g_ref, be_ref, o_ref):
    y = g_ref[...] * (h_ref[...] - mu_ref[...]) * isd_ref[...] + be_ref[...]
    o_ref[...] = jnp.where(y > 0, y, jnp.exp(jnp.minimum(y, 0.0)) - 1.0)


def _bn_elu(h, gamma, beta):
    n, c = h.shape
    nb = n // _BN_BLK
    s, q = pl.pallas_call(
        _bn_stats_kernel,
        grid=(nb,),
        in_specs=[pl.BlockSpec((_BN_BLK, c), lambda i: (i, 0))],
        out_specs=[pl.BlockSpec((1, c), lambda i: (0, 0)),
                   pl.BlockSpec((1, c), lambda i: (0, 0))],
        out_shape=[jax.ShapeDtypeStruct((1, c), h.dtype),
                   jax.ShapeDtypeStruct((1, c), h.dtype)],
    )(h)
    mu = s / n
    var = q / n - mu * mu
    isd = 1.0 / jnp.sqrt(var + 1e-5)
    return pl.pallas_call(
        _bn_apply_kernel,
        grid=(nb,),
        in_specs=[pl.BlockSpec((_BN_BLK, c), lambda i: (i, 0)),
                  pl.BlockSpec((1, c), lambda i: (0, 0)),
                  pl.BlockSpec((1, c), lambda i: (0, 0)),
                  pl.BlockSpec((1, c), lambda i: (0, 0)),
                  pl.BlockSpec((1, c), lambda i: (0, 0))],
        out_specs=pl.BlockSpec((_BN_BLK, c), lambda i: (i, 0)),
        out_shape=jax.ShapeDtypeStruct((n, c), h.dtype),
    )(h, mu, isd, gamma.reshape(1, c), beta.reshape(1, c))


def _gatv2(x, src, dst, Wl, Wr, att, bias, heads, out_ch, concat):
    n = x.shape[0]
    xl = (x @ Wl).reshape(n, heads, out_ch)
    xr = (x @ Wr).reshape(n, heads, out_ch)
    e = jax.nn.leaky_relu(xl[src] + xr[dst], negative_slope=0.2)
    logits = jnp.einsum('ehc,hc->eh', e, att)
    m = jax.ops.segment_max(logits, dst, num_segments=n)
    m = jnp.where(jnp.isfinite(m), m, 0.0)
    ex = jnp.exp(logits - m[dst])
    den = jax.ops.segment_sum(ex, dst, num_segments=n)
    alpha = ex / (den[dst] + 1e-16)
    out = jax.ops.segment_sum(xl[src] * alpha[:, :, None], dst, num_segments=n)
    if concat:
        out = out.reshape(n, heads * out_ch)
    else:
        out = out.mean(axis=1)
    return out + bias


def kernel(x, edge_index, batch, Wl1, Wr1, att1, b1, g1, be1, Wl2, Wr2, att2,
           b2, g2, be2, Wl3, Wr3, att3, b3, g3, be3):
    n = x.shape[0]
    loop = jnp.arange(n, dtype=edge_index.dtype)
    src = jnp.concatenate([edge_index[0], loop])
    dst = jnp.concatenate([edge_index[1], loop])

    h = _gatv2(x, src, dst, Wl1, Wr1, att1, b1, _H, _HID, True)
    h = _bn_elu(h, g1, be1)
    h = _gatv2(h, src, dst, Wl2, Wr2, att2, b2, _H, _HID, True)
    h = _bn_elu(h, g2, be2)
    h = _gatv2(h, src, dst, Wl3, Wr3, att3, b3, 1, _HID, False)
    h = _bn_elu(h, g3, be3)
    node_emb = h
    sums = jax.ops.segment_sum(h, batch, num_segments=_G)
    cnt = jax.ops.segment_sum(jnp.ones((n, 1), h.dtype), batch, num_segments=_G)
    graph_emb = sums / jnp.maximum(cnt, 1.0)
    return (graph_emb, node_emb)


# speedup vs baseline: 1.0128x; 1.0128x over previous
import dataclasses
import functools

import jax
import jax.numpy as jnp
from jax import lax
from jax.experimental import pallas as pl
from jax.experimental.pallas import tpu as pltpu
from jax.experimental.pallas import tpu_sc as plsc

_H = 4
_HID = 24
_G = 512
_NP = 51200
_RBLK = 6400
_NC = 2
_NS = 16
_NW = _NC * _NS
_B = 256


def _sc_params():
    return dataclasses.replace(pltpu.CompilerParams(),
                               needs_layout_passes=False,
                               use_tc_tiling_on_sc=False)


def _proj_kernel(x_ref, wl_ref, wr_ref, xl_ref, xr_ref):
    x = x_ref[...]
    xl_ref[...] = jnp.dot(x, wl_ref[...], preferred_element_type=jnp.float32)
    xr_ref[...] = jnp.dot(x, wr_ref[...], preferred_element_type=jnp.float32)


def _proj(x_pad, wlp, wrp):
    k, dp = wlp.shape
    grid = _NP // _RBLK
    return pl.pallas_call(
        _proj_kernel,
        grid=(grid,),
        in_specs=[pl.BlockSpec((_RBLK, k), lambda i: (i, 0)),
                  pl.BlockSpec((k, dp), lambda i: (0, 0)),
                  pl.BlockSpec((k, dp), lambda i: (0, 0))],
        out_specs=[pl.BlockSpec((_RBLK, dp), lambda i: (i, 0)),
                   pl.BlockSpec((_RBLK, dp), lambda i: (i, 0))],
        out_shape=[jax.ShapeDtypeStruct((_NP, dp), jnp.float32),
                   jax.ShapeDtypeStruct((_NP, dp), jnp.float32)],
    )(x_pad, wlp, wrp)


def _pad_w(w, heads):
    k = w.shape[0]
    w = w.reshape(k, heads, _HID)
    w = jnp.pad(w, ((0, 0), (0, 0), (0, 32 - _HID)))
    return w.reshape(k, heads * 32)


_BN_BLK = _RBLK


def _bn_stats_kernel(h_ref, s_ref, q_ref):
    i = pl.program_id(0)

    @pl.when(i == 0)
    def _():
        s_ref[...] = jnp.zeros_like(s_ref)
        q_ref[...] = jnp.zeros_like(q_ref)

    h = h_ref[...]
    s_ref[...] += jnp.sum(h, axis=0, keepdims=True)
    q_ref[...] += jnp.sum(h * h, axis=0, keepdims=True)


def _bn_apply_kernel(h_ref, mu_ref, isd_ref, g_ref, be_ref, o_ref):
    y = g_ref[...] * (h_ref[...] - mu_ref[...]) * isd_ref[...] + be_ref[...]
    o_ref[...] = jnp.where(y > 0, y, jnp.exp(jnp.minimum(y, 0.0)) - 1.0)


def _bn_elu(h_pad, n_real, gamma, beta):
    np_, c = h_pad.shape
    nb = np_ // _BN_BLK
    s, q = pl.pallas_call(
        _bn_stats_kernel,
        grid=(nb,),
        in_specs=[pl.BlockSpec((_BN_BLK, c), lambda i: (i, 0))],
        out_specs=[pl.BlockSpec((1, c), lambda i: (0, 0)),
                   pl.BlockSpec((1, c), lambda i: (0, 0))],
        out_shape=[jax.ShapeDtypeStruct((1, c), jnp.float32),
                   jax.ShapeDtypeStruct((1, c), jnp.float32)],
    )(h_pad)
    mu = s / n_real
    var = q / n_real - mu * mu
    isd = 1.0 / jnp.sqrt(var + 1e-5)
    return pl.pallas_call(
        _bn_apply_kernel,
        grid=(nb,),
        in_specs=[pl.BlockSpec((_BN_BLK, c), lambda i: (i, 0)),
                  pl.BlockSpec((1, c), lambda i: (0, 0)),
                  pl.BlockSpec((1, c), lambda i: (0, 0)),
                  pl.BlockSpec((1, c), lambda i: (0, 0)),
                  pl.BlockSpec((1, c), lambda i: (0, 0))],
        out_specs=pl.BlockSpec((_BN_BLK, c), lambda i: (i, 0)),
        out_shape=jax.ShapeDtypeStruct((np_, c), jnp.float32),
    )(h_pad, mu, isd, gamma.reshape(1, c), beta.reshape(1, c))


def _iota16():
    return lax.broadcasted_iota(jnp.int32, (16,), 0)


def _edge_logits_sc(xlp, xrp, src, dst, att_rep, heads, e2p):
    dp = heads * 32
    per_tile = e2p // _NW
    n_chunks = per_tile // _B
    mesh = plsc.VectorSubcoreMesh(core_axis_name="c", subcore_axis_name="s")

    @functools.partial(
        pl.kernel, mesh=mesh, compiler_params=_sc_params(),
        out_type=[jax.ShapeDtypeStruct((heads * e2p,), jnp.float32),
                  jax.ShapeDtypeStruct((_NW, 16), jnp.float32)],
        scratch_types=[
            pltpu.VMEM((_B,), jnp.int32),
            pltpu.VMEM((_B,), jnp.int32),
            pltpu.VMEM((_B, dp), jnp.float32),
            pltpu.VMEM((_B, dp), jnp.float32),
            pltpu.VMEM((heads * _B,), jnp.float32),
            pltpu.VMEM((heads * _HID * 16,), jnp.float32),
            pltpu.VMEM((16,), jnp.float32),
            pltpu.SemaphoreType.DMA,
            pltpu.SemaphoreType.DMA,
        ],
    )
    def kern(xl_hbm, xr_hbm, src_hbm, dst_hbm, att_hbm, lo_hbm, mx_hbm,
             srcv, dstv, xlr, xrr, lchunk, attv, mxv, sem1, sem2):
        wid = lax.axis_index("s") * _NC + lax.axis_index("c")
        tbase = wid * per_tile
        pltpu.sync_copy(att_hbm, attv)
        mxv[...] = jnp.full((16,), -3e38, jnp.float32)

        @pl.loop(0, n_chunks)
        def _chunk(ci):
            base = tbase + ci * _B
            pltpu.sync_copy(src_hbm.at[pl.ds(base, _B)], srcv)
            pltpu.sync_copy(dst_hbm.at[pl.ds(base, _B)], dstv)
            cp1 = pltpu.async_copy(xl_hbm.at[srcv], xlr, sem1)
            cp2 = pltpu.async_copy(xr_hbm.at[dstv], xrr, sem2)
            cp1.wait()
            cp2.wait()

            @pl.loop(0, _B // 16)
            def _group(g):
                ev = _iota16() + g * 16
                for h in range(heads):
                    acc = jnp.zeros((16,), jnp.float32)
                    for c in range(_HID):
                        cv = jnp.full((16,), h * 32 + c, jnp.int32)
                        a = plsc.load_gather(xlr, [ev, cv])
                        b = plsc.load_gather(xrr, [ev, cv])
                        z = a + b
                        lr = jnp.maximum(z, 0.0) + 0.2 * jnp.minimum(z, 0.0)
                        av = attv[pl.ds((h * _HID + c) * 16, 16)]
                        acc = acc + lr * av
                    lchunk[pl.ds(h * _B + g * 16, 16)] = acc
                    mxv[...] = jnp.maximum(mxv[...], acc)

            for h in range(heads):
                pltpu.sync_copy(lchunk.at[pl.ds(h * _B, _B)],
                                lo_hbm.at[pl.ds(h * e2p + base, _B)])

        pltpu.sync_copy(mxv, mx_hbm.at[wid])

    return kern(xlp, xrp, src, dst, att_rep)


def _gatv2_layer(h_pad, src, dst, wl, wr, att, bias, heads, e2, e2p):
    n = 50000
    dp = heads * 32
    wlp = _pad_w(wl, heads)
    wrp = _pad_w(wr, heads)
    xlp, xrp = _proj(h_pad, wlp, wrp)
    att_rep = jnp.repeat(att.reshape(-1), 16)

    lo_flat, mx = _edge_logits_sc(xlp, xrp, src, dst, att_rep, heads, e2p)
    logits = lo_flat.reshape(heads, e2p)[:, :e2].T

    m = jnp.max(mx)
    ex = jnp.exp(logits - m)
    den = jax.ops.segment_sum(ex, dst[:e2], num_segments=n)
    alpha = ex / (den[dst[:e2]] + 1e-16)
    xl = xlp[:n].reshape(n, heads, 32)[:, :, :_HID]
    out = jax.ops.segment_sum(
        xl[src[:e2]] * alpha[:, :, None], dst[:e2], num_segments=n)
    out = out.reshape(n, heads * _HID) if heads > 1 else out[:, 0]
    return out + bias


def kernel(x, edge_index, batch, Wl1, Wr1, att1, b1, g1, be1, Wl2, Wr2, att2,
           b2, g2, be2, Wl3, Wr3, att3, b3, g3, be3):
    n = 50000
    e = edge_index.shape[1]
    e2 = e + n
    e2p = ((e2 + _NW * _B - 1) // (_NW * _B)) * (_NW * _B)

    loop = jnp.arange(n, dtype=jnp.int32)
    src0 = jnp.concatenate([edge_index[0], loop,
                            jnp.zeros((e2p - e2,), jnp.int32)])
    dst0 = jnp.concatenate([edge_index[1], loop,
                            jnp.full((e2p - e2,), n, jnp.int32)])
    order = jnp.argsort(dst0[:e2])
    src = jnp.concatenate([src0[order], src0[e2:]])
    dst = jnp.concatenate([dst0[order], dst0[e2:]])

    def pad_rows(h):
        return jnp.pad(h, ((0, _NP - h.shape[0]), (0, 0)))

    h = pad_rows(x.astype(jnp.float32))
    h = jnp.pad(h, ((0, 0), (0, 2)))
    h = _gatv2_layer(h, src, dst, jnp.pad(Wl1, ((0, 2), (0, 0))),
                     jnp.pad(Wr1, ((0, 2), (0, 0))), att1, b1, _H, e2, e2p)
    h = _bn_elu(pad_rows(h), n, g1, be1)
    h2 = _gatv2_layer(h, src, dst, Wl2, Wr2, att2, b2, _H, e2, e2p)
    h = _bn_elu(pad_rows(h2), n, g2, be2)
    h3 = _gatv2_layer(h, src, dst, Wl3, Wr3, att3, b3, 1, e2, e2p)
    h = _bn_elu(pad_rows(h3), n, g3, be3)[:n]

    node_emb = h
    sums = jax.ops.segment_sum(h, batch, num_segments=_G)
    cnt = jax.ops.segment_sum(jnp.ones((n, 1), h.dtype), batch,
                              num_segments=_G)
    graph_emb = sums / jnp.maximum(cnt, 1.0)
    return (graph_emb, node_emb)


# speedup vs baseline: 13.0914x; 12.9256x over previous
import dataclasses
import functools

import jax
import jax.numpy as jnp
from jax import lax
from jax.experimental import pallas as pl
from jax.experimental.pallas import tpu as pltpu
from jax.experimental.pallas import tpu_sc as plsc

_H = 4
_HID = 24
_G = 512
_NP = 51200
_RBLK = 6400
_NC = 2
_NS = 16
_NW = _NC * _NS
_B = 256


def _sc_params():
    return dataclasses.replace(pltpu.CompilerParams(),
                               needs_layout_passes=False,
                               use_tc_tiling_on_sc=False)


def _proj_kernel(x_ref, wl_ref, wr_ref, xl_ref, xr_ref):
    x = x_ref[...]
    xl_ref[...] = jnp.dot(x, wl_ref[...], preferred_element_type=jnp.float32)
    xr_ref[...] = jnp.dot(x, wr_ref[...], preferred_element_type=jnp.float32)


def _proj(x_pad, wlp, wrp):
    k, dp = wlp.shape
    grid = _NP // _RBLK
    return pl.pallas_call(
        _proj_kernel,
        grid=(grid,),
        in_specs=[pl.BlockSpec((_RBLK, k), lambda i: (i, 0)),
                  pl.BlockSpec((k, dp), lambda i: (0, 0)),
                  pl.BlockSpec((k, dp), lambda i: (0, 0))],
        out_specs=[pl.BlockSpec((_RBLK, dp), lambda i: (i, 0)),
                   pl.BlockSpec((_RBLK, dp), lambda i: (i, 0))],
        out_shape=[jax.ShapeDtypeStruct((_NP, dp), jnp.float32),
                   jax.ShapeDtypeStruct((_NP, dp), jnp.float32)],
    )(x_pad, wlp, wrp)


def _pad_w(w, heads):
    k = w.shape[0]
    w = w.reshape(k, heads, _HID)
    w = jnp.pad(w, ((0, 0), (0, 0), (0, 32 - _HID)))
    return w.reshape(k, heads * 32)


_BN_BLK = _RBLK


def _bn_stats_kernel(h_ref, s_ref, q_ref):
    i = pl.program_id(0)

    @pl.when(i == 0)
    def _():
        s_ref[...] = jnp.zeros_like(s_ref)
        q_ref[...] = jnp.zeros_like(q_ref)

    h = h_ref[...]
    s_ref[...] += jnp.sum(h, axis=0, keepdims=True)
    q_ref[...] += jnp.sum(h * h, axis=0, keepdims=True)


def _bn_apply_kernel(h_ref, mu_ref, isd_ref, g_ref, be_ref, o_ref):
    y = g_ref[...] * (h_ref[...] - mu_ref[...]) * isd_ref[...] + be_ref[...]
    o_ref[...] = jnp.where(y > 0, y, jnp.exp(jnp.minimum(y, 0.0)) - 1.0)


def _bn_elu(h_pad, n_real, gamma, beta):
    np_, c = h_pad.shape
    nb = np_ // _BN_BLK
    s, q = pl.pallas_call(
        _bn_stats_kernel,
        grid=(nb,),
        in_specs=[pl.BlockSpec((_BN_BLK, c), lambda i: (i, 0))],
        out_specs=[pl.BlockSpec((1, c), lambda i: (0, 0)),
                   pl.BlockSpec((1, c), lambda i: (0, 0))],
        out_shape=[jax.ShapeDtypeStruct((1, c), jnp.float32),
                   jax.ShapeDtypeStruct((1, c), jnp.float32)],
    )(h_pad)
    mu = s / n_real
    var = q / n_real - mu * mu
    isd = 1.0 / jnp.sqrt(var + 1e-5)
    return pl.pallas_call(
        _bn_apply_kernel,
        grid=(nb,),
        in_specs=[pl.BlockSpec((_BN_BLK, c), lambda i: (i, 0)),
                  pl.BlockSpec((1, c), lambda i: (0, 0)),
                  pl.BlockSpec((1, c), lambda i: (0, 0)),
                  pl.BlockSpec((1, c), lambda i: (0, 0)),
                  pl.BlockSpec((1, c), lambda i: (0, 0))],
        out_specs=pl.BlockSpec((_BN_BLK, c), lambda i: (i, 0)),
        out_shape=jax.ShapeDtypeStruct((np_, c), jnp.float32),
    )(h_pad, mu, isd, gamma.reshape(1, c), beta.reshape(1, c))


def _iota16():
    return lax.broadcasted_iota(jnp.int32, (16,), 0)


def _edge_logits_sc(xlp, xrp, src, dst, att_rep, heads, e2p):
    dp = heads * 32
    per_tile = e2p // _NW
    n_chunks = per_tile // _B
    mesh = plsc.VectorSubcoreMesh(core_axis_name="c", subcore_axis_name="s")

    @functools.partial(
        pl.kernel, mesh=mesh, compiler_params=_sc_params(),
        out_type=[jax.ShapeDtypeStruct((heads * e2p,), jnp.float32),
                  jax.ShapeDtypeStruct((_NW, 16), jnp.float32)],
        scratch_types=[
            pltpu.VMEM((_B,), jnp.int32),
            pltpu.VMEM((_B,), jnp.int32),
            pltpu.VMEM((_B, dp), jnp.float32),
            pltpu.VMEM((_B, dp), jnp.float32),
            pltpu.VMEM((heads * _B,), jnp.float32),
            pltpu.VMEM((heads * _HID * 16,), jnp.float32),
            pltpu.VMEM((16,), jnp.float32),
            pltpu.SemaphoreType.DMA,
            pltpu.SemaphoreType.DMA,
        ],
    )
    def kern(xl_hbm, xr_hbm, src_hbm, dst_hbm, att_hbm, lo_hbm, mx_hbm,
             srcv, dstv, xlr, xrr, lchunk, attv, mxv, sem1, sem2):
        wid = lax.axis_index("s") * _NC + lax.axis_index("c")
        tbase = wid * per_tile
        pltpu.sync_copy(att_hbm, attv)
        mxv[...] = jnp.full((16,), -3e38, jnp.float32)

        @pl.loop(0, n_chunks)
        def _chunk(ci):
            base = tbase + ci * _B
            pltpu.sync_copy(src_hbm.at[pl.ds(base, _B)], srcv)
            pltpu.sync_copy(dst_hbm.at[pl.ds(base, _B)], dstv)
            cp1 = pltpu.async_copy(xl_hbm.at[srcv], xlr, sem1)
            cp2 = pltpu.async_copy(xr_hbm.at[dstv], xrr, sem2)
            cp1.wait()
            cp2.wait()

            @pl.loop(0, _B // 16)
            def _group(g):
                ev = _iota16() + g * 16
                for h in range(heads):
                    acc = jnp.zeros((16,), jnp.float32)
                    for c in range(_HID):
                        cv = jnp.full((16,), h * 32 + c, jnp.int32)
                        a = plsc.load_gather(xlr, [ev, cv])
                        b = plsc.load_gather(xrr, [ev, cv])
                        z = a + b
                        lr = jnp.maximum(z, 0.0) + 0.2 * jnp.minimum(z, 0.0)
                        av = attv[pl.ds((h * _HID + c) * 16, 16)]
                        acc = acc + lr * av
                    lchunk[pl.ds(h * _B + g * 16, 16)] = acc
                    mxv[...] = jnp.maximum(mxv[...], acc)

            for h in range(heads):
                pltpu.sync_copy(lchunk.at[pl.ds(h * _B, _B)],
                                lo_hbm.at[pl.ds(h * e2p + base, _B)])

        pltpu.sync_copy(mxv, mx_hbm.at[wid])

    return kern(xlp, xrp, src, dst, att_rep)


_DN = 50176
_DSTRIPE = _DN // _NS


def _den_sc(lo_flat, dst, mv, heads, e2p):
    per_tile = e2p // _NW
    n_chunks = per_tile // _B
    mesh = plsc.VectorSubcoreMesh(core_axis_name="c", subcore_axis_name="s")

    @functools.partial(
        pl.kernel, mesh=mesh, compiler_params=_sc_params(),
        out_type=jax.ShapeDtypeStruct((_NC, _DN, 16), jnp.float32),
        scratch_types=[
            pltpu.VMEM((_B,), jnp.int32),
            pltpu.VMEM((heads * _B,), jnp.float32),
            pltpu.VMEM((_B, 16), jnp.float32),
            pltpu.VMEM((16,), jnp.float32),
            pltpu.VMEM_SHARED((_DN, 16), jnp.float32),
        ],
    )
    def kern(lo_hbm, dst_hbm, mv_hbm, zer_hbm, den_hbm, dstv, lhv, exr, mvv,
             den_sp):
        core = lax.axis_index("c")
        sid = lax.axis_index("s")
        wid = sid * _NC + core
        tbase = wid * per_tile
        pltpu.sync_copy(mv_hbm, mvv)
        pltpu.sync_copy(zer_hbm, exr)

        @pl.loop(0, _DSTRIPE // _B)
        def _zs(i):
            pltpu.sync_copy(exr, den_sp.at[pl.ds(sid * _DSTRIPE + i * _B, _B)])

        rem = _DSTRIPE % _B
        if rem:
            pltpu.sync_copy(exr.at[pl.ds(0, rem)],
                            den_sp.at[pl.ds(sid * _DSTRIPE
                                            + (_DSTRIPE // _B) * _B, rem)])
        plsc.subcore_barrier()

        @pl.loop(0, n_chunks)
        def _chunk(ci):
            base = tbase + ci * _B
            pltpu.sync_copy(dst_hbm.at[pl.ds(base, _B)], dstv)
            for h in range(heads):
                pltpu.sync_copy(lo_hbm.at[pl.ds(h * e2p + base, _B)],
                                lhv.at[pl.ds(h * _B, _B)])

            @pl.loop(0, _B // 16)
            def _group(g):
                ev = _iota16() + g * 16
                for h in range(heads):
                    l = lhv[pl.ds(h * _B + g * 16, 16)]
                    ex = jnp.exp(l - mvv[...])
                    plsc.store_scatter(exr, [ev, jnp.full((16,), h, jnp.int32)],
                                       ex)
            pltpu.sync_copy(exr, den_sp.at[dstv], add=True)

        plsc.subcore_barrier()
        pltpu.sync_copy(den_sp.at[pl.ds(sid * _DSTRIPE, _DSTRIPE)],
                        den_hbm.at[core, pl.ds(sid * _DSTRIPE, _DSTRIPE)])

    return kern(lo_flat, dst, mv, jnp.zeros((_B, 16), jnp.float32))


_QN = 6250
_NQ = 8
_AROWS = 6272
_ASTRIPE = _AROWS // _NS


def _extract_i32(vec, i):
    return jnp.sum(jnp.where(_iota16() == i, vec, 0))


def _aggregate_sc(xlp, src, dst, lo_flat, mv, invd, qs, heads, e2p):
    dp = heads * 32
    dmsg = 96 if heads > 1 else 32
    mesh = plsc.VectorSubcoreMesh(core_axis_name="c", subcore_axis_name="s")

    @functools.partial(
        pl.kernel, mesh=mesh, compiler_params=_sc_params(),
        out_type=jax.ShapeDtypeStruct((_NQ, _AROWS, dmsg), jnp.float32),
        scratch_types=[
            pltpu.VMEM((_B,), jnp.int32),
            pltpu.VMEM((_B,), jnp.int32),
            pltpu.VMEM((_B,), jnp.int32),
            pltpu.VMEM((_B, dp), jnp.float32),
            pltpu.VMEM((_B, dmsg), jnp.float32),
            pltpu.VMEM((heads * _B,), jnp.float32),
            pltpu.VMEM((_B, 16), jnp.float32),
            pltpu.VMEM((16,), jnp.float32),
            pltpu.VMEM((16,), jnp.int32),
            pltpu.VMEM_SHARED((_AROWS, dmsg), jnp.float32),
            pltpu.SemaphoreType.DMA,
            pltpu.SemaphoreType.DMA,
        ],
    )
    def kern(xl_hbm, src_hbm, dst_hbm, lo_hbm, mv_hbm, inv_hbm, qs_hbm,
             zer_hbm, out_hbm, srcv, dstv, dloc, xlr, msg, lhv, invr, mvv,
             qsv, acc_sp, sem1, sem2):
        core = lax.axis_index("c")
        sid = lax.axis_index("s")
        pltpu.sync_copy(mv_hbm, mvv)
        pltpu.sync_copy(qs_hbm, qsv)
        pltpu.sync_copy(zer_hbm, msg)

        for j in range(_NQ // 2):
            q = core * (_NQ // 2) + j
            qsvv = qsv[...]
            qlo = _extract_i32(qsvv, q)
            qhi = _extract_i32(qsvv, q + 1)
            qbase = q * _QN
            per_t = (qhi - qlo + _NS - 1) // _NS
            s_k = qlo + sid * per_t
            e_k = jnp.minimum(s_k + per_t, qhi)
            s8 = (s_k // 8) * 8
            nch = jnp.maximum((e_k - s8 + _B - 1) // _B, 0)

            @pl.loop(0, _ASTRIPE // _B)
            def _za(i):
                pltpu.sync_copy(msg, acc_sp.at[pl.ds(sid * _ASTRIPE + i * _B,
                                                     _B)])

            rem = _ASTRIPE % _B
            if rem:
                pltpu.sync_copy(msg.at[pl.ds(0, rem)],
                                acc_sp.at[pl.ds(sid * _ASTRIPE
                                                + (_ASTRIPE // _B) * _B, rem)])
            plsc.subcore_barrier()

            def _chunk(ci, carry):
                base = s8 + ci * _B
                pltpu.sync_copy(src_hbm.at[pl.ds(base, _B)], srcv)
                pltpu.sync_copy(dst_hbm.at[pl.ds(base, _B)], dstv)
                cp1 = pltpu.async_copy(xl_hbm.at[srcv], xlr, sem1)
                cp2 = pltpu.async_copy(inv_hbm.at[dstv], invr, sem2)
                for h in range(heads):
                    pltpu.sync_copy(lo_hbm.at[pl.ds(h * e2p + base, _B)],
                                    lhv.at[pl.ds(h * _B, _B)])
                cp1.wait()
                cp2.wait()

                @pl.loop(0, _B // 16)
                def _group(g):
                    ev = _iota16() + g * 16
                    eg = base + ev
                    inq = (eg >= s_k) & (eg < e_k)
                    dv = dstv[pl.ds(g * 16, 16)]
                    dloc[pl.ds(g * 16, 16)] = jnp.where(inq, dv - qbase, _QN)
                    for h in range(heads):
                        l = lhv[pl.ds(h * _B + g * 16, 16)]
                        ex = jnp.exp(l - mvv[...])
                        iv = plsc.load_gather(
                            invr, [ev, jnp.full((16,), h, jnp.int32)])
                        alpha = ex * iv
                        for c in range(_HID):
                            xv = plsc.load_gather(
                                xlr, [ev, jnp.full((16,), h * 32 + c,
                                                   jnp.int32)])
                            plsc.store_scatter(
                                msg, [ev, jnp.full((16,), h * _HID + c,
                                                   jnp.int32)],
                                xv * alpha)

                pltpu.sync_copy(msg, acc_sp.at[dloc], add=True)
                return carry

            lax.fori_loop(0, nch, _chunk, 0)
            plsc.subcore_barrier()

            @pl.loop(0, _ASTRIPE // _B)
            def _fl(i):
                pltpu.sync_copy(acc_sp.at[pl.ds(sid * _ASTRIPE + i * _B, _B)],
                                out_hbm.at[q, pl.ds(sid * _ASTRIPE + i * _B,
                                                    _B)])

            if rem:
                pltpu.sync_copy(
                    acc_sp.at[pl.ds(sid * _ASTRIPE + (_ASTRIPE // _B) * _B,
                                    rem)],
                    out_hbm.at[q, pl.ds(sid * _ASTRIPE + (_ASTRIPE // _B) * _B,
                                        rem)])

            if j < _NQ // 2 - 1:
                pltpu.sync_copy(zer_hbm, msg)

    return kern(xlp, src, dst, lo_flat, mv, invd, qs,
                jnp.zeros((_B, dmsg), jnp.float32))


_PB = 320
_PROWS = 528
_PSTRIPE = _PROWS // _NS


def _pool_sc(h_pad, batch_pad):
    per_tile = _NP // _NW
    n_chunks = per_tile // _PB
    mesh = plsc.VectorSubcoreMesh(core_axis_name="c", subcore_axis_name="s")

    @functools.partial(
        pl.kernel, mesh=mesh, compiler_params=_sc_params(),
        out_type=[jax.ShapeDtypeStruct((_NC, _PROWS, 32), jnp.float32),
                  jax.ShapeDtypeStruct((_NC, _PROWS, 32), jnp.float32)],
        scratch_types=[
            pltpu.VMEM((_PB,), jnp.int32),
            pltpu.VMEM((_PB, 32), jnp.float32),
            pltpu.VMEM((_PB, 32), jnp.float32),
            pltpu.VMEM_SHARED((_PROWS, 32), jnp.float32),
            pltpu.VMEM_SHARED((_PROWS, 32), jnp.float32),
        ],
    )
    def kern(h_hbm, b_hbm, ones_hbm, zer_hbm, sum_hbm, cnt_hbm, bv, hrows,
             ones, sum_sp, cnt_sp):
        core = lax.axis_index("c")
        sid = lax.axis_index("s")
        wid = sid * _NC + core
        tbase = wid * per_tile
        pltpu.sync_copy(ones_hbm, ones)
        pltpu.sync_copy(zer_hbm, hrows)

        pltpu.sync_copy(hrows.at[pl.ds(0, _PSTRIPE)],
                        sum_sp.at[pl.ds(sid * _PSTRIPE, _PSTRIPE)])
        pltpu.sync_copy(hrows.at[pl.ds(0, _PSTRIPE)],
                        cnt_sp.at[pl.ds(sid * _PSTRIPE, _PSTRIPE)])
        plsc.subcore_barrier()

        @pl.loop(0, n_chunks)
        def _chunk(ci):
            base = tbase + ci * _PB
            pltpu.sync_copy(b_hbm.at[pl.ds(base, _PB)], bv)
            pltpu.sync_copy(h_hbm.at[pl.ds(base, _PB)], hrows)
            pltpu.sync_copy(hrows, sum_sp.at[bv], add=True)
            pltpu.sync_copy(ones, cnt_sp.at[bv], add=True)

        plsc.subcore_barrier()
        pltpu.sync_copy(sum_sp.at[pl.ds(sid * _PSTRIPE, _PSTRIPE)],
                        sum_hbm.at[core, pl.ds(sid * _PSTRIPE, _PSTRIPE)])
        pltpu.sync_copy(cnt_sp.at[pl.ds(sid * _PSTRIPE, _PSTRIPE)],
                        cnt_hbm.at[core, pl.ds(sid * _PSTRIPE, _PSTRIPE)])

    return kern(h_pad, batch_pad, jnp.ones((_PB, 32), jnp.float32),
                jnp.zeros((_PB, 32), jnp.float32))


def _gatv2_layer(h_pad, src, dst, qs, wl, wr, att, bias, heads, e2, e2p):
    wlp = _pad_w(wl, heads)
    wrp = _pad_w(wr, heads)
    xlp, xrp = _proj(h_pad, wlp, wrp)
    att_rep = jnp.repeat(att.reshape(-1), 16)

    lo_flat, mx = _edge_logits_sc(xlp, xrp, src, dst, att_rep, heads, e2p)
    m = jnp.max(mx)
    mv = jnp.full((16,), m, jnp.float32)
    den2 = _den_sc(lo_flat, dst, mv, heads, e2p)
    invd = 1.0 / (den2[0] + den2[1] + 1e-16)
    out4 = _aggregate_sc(xlp, src, dst, lo_flat, mv, invd, qs, heads, e2p)
    d = heads * _HID
    out = out4[:, :_QN, :d].reshape(_NQ * _QN, d)
    return out + bias


def kernel(x, edge_index, batch, Wl1, Wr1, att1, b1, g1, be1, Wl2, Wr2, att2,
           b2, g2, be2, Wl3, Wr3, att3, b3, g3, be3):
    n = 50000
    e = edge_index.shape[1]
    e2 = e + n
    e2p = ((e2 + _NW * _B - 1) // (_NW * _B)) * (_NW * _B)

    loop = jnp.arange(n, dtype=jnp.int32)
    src0 = jnp.concatenate([edge_index[0], loop,
                            jnp.zeros((e2p - e2,), jnp.int32)])
    dst0 = jnp.concatenate([edge_index[1], loop,
                            jnp.full((e2p - e2,), n, jnp.int32)])
    order = jnp.argsort(dst0[:e2])
    src = jnp.concatenate([src0[order], src0[e2:]])
    dst = jnp.concatenate([dst0[order], dst0[e2:]])
    qs = jnp.searchsorted(
        dst, jnp.arange(0, (_NQ + 1) * _QN, _QN, dtype=jnp.int32))
    qs = jnp.concatenate([qs.astype(jnp.int32),
                          jnp.zeros((16 - _NQ - 1,), jnp.int32)])

    def pad_rows(h):
        return jnp.pad(h, ((0, _NP - h.shape[0]), (0, 0)))

    h = pad_rows(x.astype(jnp.float32))
    h = jnp.pad(h, ((0, 0), (0, 2)))
    h = _gatv2_layer(h, src, dst, qs, jnp.pad(Wl1, ((0, 2), (0, 0))),
                     jnp.pad(Wr1, ((0, 2), (0, 0))), att1, b1, _H, e2, e2p)
    h = _bn_elu(pad_rows(h), n, g1, be1)
    h = _gatv2_layer(h, src, dst, qs, Wl2, Wr2, att2, b2, _H, e2, e2p)
    h = _bn_elu(pad_rows(h), n, g2, be2)
    h = _gatv2_layer(h, src, dst, qs, Wl3, Wr3, att3, b3, 1, e2, e2p)
    h = _bn_elu(pad_rows(h), n, g3, be3)

    node_emb = h[:n]
    h32 = jnp.pad(h, ((0, 0), (0, 8)))
    batch_pad = jnp.concatenate([batch.astype(jnp.int32),
                                 jnp.full((_NP - n,), _G, jnp.int32)])
    sums2, cnts2 = _pool_sc(h32, batch_pad)
    sums = (sums2[0] + sums2[1])[:_G, :_HID]
    cnt = (cnts2[0] + cnts2[1])[:_G, :1]
    graph_emb = sums / jnp.maximum(cnt, 1.0)
    return (graph_emb, node_emb)


# speedup vs baseline: 13.7764x; 1.0523x over previous
import dataclasses
import functools

import jax
import jax.numpy as jnp
from jax import lax
from jax.experimental import pallas as pl
from jax.experimental.pallas import tpu as pltpu
from jax.experimental.pallas import tpu_sc as plsc

_H = 4
_HID = 24
_G = 512
_NP = 51200
_RBLK = 6400
_NC = 2
_NS = 16
_NW = _NC * _NS
_B = 256


def _sc_params():
    return dataclasses.replace(pltpu.CompilerParams(),
                               needs_layout_passes=False,
                               use_tc_tiling_on_sc=False)


def _proj_kernel(x_ref, wl_ref, wr_ref, xl_ref, xr_ref):
    x = x_ref[...]
    xl_ref[...] = jnp.dot(x, wl_ref[...], preferred_element_type=jnp.float32)
    xr_ref[...] = jnp.dot(x, wr_ref[...], preferred_element_type=jnp.float32)


def _proj(x_pad, wlp, wrp):
    k, dp = wlp.shape
    grid = _NP // _RBLK
    return pl.pallas_call(
        _proj_kernel,
        grid=(grid,),
        in_specs=[pl.BlockSpec((_RBLK, k), lambda i: (i, 0)),
                  pl.BlockSpec((k, dp), lambda i: (0, 0)),
                  pl.BlockSpec((k, dp), lambda i: (0, 0))],
        out_specs=[pl.BlockSpec((_RBLK, dp), lambda i: (i, 0)),
                   pl.BlockSpec((_RBLK, dp), lambda i: (i, 0))],
        out_shape=[jax.ShapeDtypeStruct((_NP, dp), jnp.float32),
                   jax.ShapeDtypeStruct((_NP, dp), jnp.float32)],
    )(x_pad, wlp, wrp)


def _pad_w(w, heads):
    k = w.shape[0]
    w = w.reshape(k, heads, _HID)
    w = jnp.pad(w, ((0, 0), (0, 0), (0, 32 - _HID)))
    return w.reshape(k, heads * 32)


_BN_BLK = _RBLK


def _bn_stats_kernel(h_ref, s_ref, q_ref):
    i = pl.program_id(0)

    @pl.when(i == 0)
    def _():
        s_ref[...] = jnp.zeros_like(s_ref)
        q_ref[...] = jnp.zeros_like(q_ref)

    h = h_ref[...]
    s_ref[...] += jnp.sum(h, axis=0, keepdims=True)
    q_ref[...] += jnp.sum(h * h, axis=0, keepdims=True)


def _bn_apply_kernel(h_ref, mu_ref, isd_ref, g_ref, be_ref, o_ref):
    y = g_ref[...] * (h_ref[...] - mu_ref[...]) * isd_ref[...] + be_ref[...]
    o_ref[...] = jnp.where(y > 0, y, jnp.exp(jnp.minimum(y, 0.0)) - 1.0)


def _bn_elu(h_pad, n_real, gamma, beta):
    np_, c = h_pad.shape
    nb = np_ // _BN_BLK
    s, q = pl.pallas_call(
        _bn_stats_kernel,
        grid=(nb,),
        in_specs=[pl.BlockSpec((_BN_BLK, c), lambda i: (i, 0))],
        out_specs=[pl.BlockSpec((1, c), lambda i: (0, 0)),
                   pl.BlockSpec((1, c), lambda i: (0, 0))],
        out_shape=[jax.ShapeDtypeStruct((1, c), jnp.float32),
                   jax.ShapeDtypeStruct((1, c), jnp.float32)],
    )(h_pad)
    mu = s / n_real
    var = q / n_real - mu * mu
    isd = 1.0 / jnp.sqrt(var + 1e-5)
    return pl.pallas_call(
        _bn_apply_kernel,
        grid=(nb,),
        in_specs=[pl.BlockSpec((_BN_BLK, c), lambda i: (i, 0)),
                  pl.BlockSpec((1, c), lambda i: (0, 0)),
                  pl.BlockSpec((1, c), lambda i: (0, 0)),
                  pl.BlockSpec((1, c), lambda i: (0, 0)),
                  pl.BlockSpec((1, c), lambda i: (0, 0))],
        out_specs=pl.BlockSpec((_BN_BLK, c), lambda i: (i, 0)),
        out_shape=jax.ShapeDtypeStruct((np_, c), jnp.float32),
    )(h_pad, mu, isd, gamma.reshape(1, c), beta.reshape(1, c))


def _iota16():
    return lax.broadcasted_iota(jnp.int32, (16,), 0)


def _edge_logits_sc(xlp, xrp, src, dst, att_rep, heads, e2p):
    dp = heads * 32
    per_tile = e2p // _NW
    n_chunks = per_tile // _B
    mesh = plsc.VectorSubcoreMesh(core_axis_name="c", subcore_axis_name="s")

    @functools.partial(
        pl.kernel, mesh=mesh, compiler_params=_sc_params(),
        out_type=[jax.ShapeDtypeStruct((heads * e2p,), jnp.float32),
                  jax.ShapeDtypeStruct((_NW, 16), jnp.float32)],
        scratch_types=[
            pltpu.VMEM((_B,), jnp.int32),
            pltpu.VMEM((_B,), jnp.int32),
            pltpu.VMEM((_B, dp), jnp.float32),
            pltpu.VMEM((_B, dp), jnp.float32),
            pltpu.VMEM((heads * _B,), jnp.float32),
            pltpu.VMEM((heads * _HID * 16,), jnp.float32),
            pltpu.VMEM((16,), jnp.float32),
            pltpu.SemaphoreType.DMA,
            pltpu.SemaphoreType.DMA,
        ],
    )
    def kern(xl_hbm, xr_hbm, src_hbm, dst_hbm, att_hbm, lo_hbm, mx_hbm,
             srcv, dstv, xlr, xrr, lchunk, attv, mxv, sem1, sem2):
        wid = lax.axis_index("s") * _NC + lax.axis_index("c")
        tbase = wid * per_tile
        pltpu.sync_copy(att_hbm, attv)
        mxv[...] = jnp.full((16,), -3e38, jnp.float32)

        @pl.loop(0, n_chunks)
        def _chunk(ci):
            base = tbase + ci * _B
            pltpu.sync_copy(src_hbm.at[pl.ds(base, _B)], srcv)
            pltpu.sync_copy(dst_hbm.at[pl.ds(base, _B)], dstv)
            cp1 = pltpu.async_copy(xl_hbm.at[srcv], xlr, sem1)
            cp2 = pltpu.async_copy(xr_hbm.at[dstv], xrr, sem2)
            cp1.wait()
            cp2.wait()

            @plsc.parallel_loop(0, _B // 16, unroll=2,
                                carry=jnp.full((16,), -3e38, jnp.float32))
            def _group(g, mxc):
                ev = _iota16() + g * 16
                for h in range(heads):
                    acc = jnp.zeros((16,), jnp.float32)
                    for c in range(_HID):
                        cv = jnp.full((16,), h * 32 + c, jnp.int32)
                        a = plsc.load_gather(xlr, [ev, cv])
                        b = plsc.load_gather(xrr, [ev, cv])
                        z = a + b
                        lr = jnp.maximum(z, 0.0) + 0.2 * jnp.minimum(z, 0.0)
                        av = attv[pl.ds((h * _HID + c) * 16, 16)]
                        acc = acc + lr * av
                    lchunk[pl.ds(h * _B + g * 16, 16)] = acc
                    mxc = jnp.maximum(mxc, acc)
                return mxc

            mxv[...] = jnp.maximum(mxv[...], _group)

            for h in range(heads):
                pltpu.sync_copy(lchunk.at[pl.ds(h * _B, _B)],
                                lo_hbm.at[pl.ds(h * e2p + base, _B)])

        pltpu.sync_copy(mxv, mx_hbm.at[wid])

    return kern(xlp, xrp, src, dst, att_rep)


_DN = 50176
_DSTRIPE = _DN // _NS


def _den_sc(lo_flat, dst, mv, heads, e2p):
    per_tile = e2p // _NW
    n_chunks = per_tile // _B
    mesh = plsc.VectorSubcoreMesh(core_axis_name="c", subcore_axis_name="s")

    @functools.partial(
        pl.kernel, mesh=mesh, compiler_params=_sc_params(),
        out_type=jax.ShapeDtypeStruct((_NC, _DN, 16), jnp.float32),
        scratch_types=[
            pltpu.VMEM((_B,), jnp.int32),
            pltpu.VMEM((heads * _B,), jnp.float32),
            pltpu.VMEM((_B, 16), jnp.float32),
            pltpu.VMEM((16,), jnp.float32),
            pltpu.VMEM_SHARED((_DN, 16), jnp.float32),
        ],
    )
    def kern(lo_hbm, dst_hbm, mv_hbm, zer_hbm, den_hbm, dstv, lhv, exr, mvv,
             den_sp):
        core = lax.axis_index("c")
        sid = lax.axis_index("s")
        wid = sid * _NC + core
        tbase = wid * per_tile
        pltpu.sync_copy(mv_hbm, mvv)
        pltpu.sync_copy(zer_hbm, exr)

        @pl.loop(0, _DSTRIPE // _B)
        def _zs(i):
            pltpu.sync_copy(exr, den_sp.at[pl.ds(sid * _DSTRIPE + i * _B, _B)])

        rem = _DSTRIPE % _B
        if rem:
            pltpu.sync_copy(exr.at[pl.ds(0, rem)],
                            den_sp.at[pl.ds(sid * _DSTRIPE
                                            + (_DSTRIPE // _B) * _B, rem)])
        plsc.subcore_barrier()

        @pl.loop(0, n_chunks)
        def _chunk(ci):
            base = tbase + ci * _B
            pltpu.sync_copy(dst_hbm.at[pl.ds(base, _B)], dstv)
            for h in range(heads):
                pltpu.sync_copy(lo_hbm.at[pl.ds(h * e2p + base, _B)],
                                lhv.at[pl.ds(h * _B, _B)])

            @plsc.parallel_loop(0, _B // 16, unroll=2)
            def _group(g):
                ev = _iota16() + g * 16
                for h in range(heads):
                    l = lhv[pl.ds(h * _B + g * 16, 16)]
                    ex = jnp.exp(l - mvv[...])
                    plsc.store_scatter(exr, [ev, jnp.full((16,), h, jnp.int32)],
                                       ex)
            pltpu.sync_copy(exr, den_sp.at[dstv], add=True)

        plsc.subcore_barrier()
        pltpu.sync_copy(den_sp.at[pl.ds(sid * _DSTRIPE, _DSTRIPE)],
                        den_hbm.at[core, pl.ds(sid * _DSTRIPE, _DSTRIPE)])

    return kern(lo_flat, dst, mv, jnp.zeros((_B, 16), jnp.float32))


_QN = 6250
_NQ = 8
_AROWS = 6272
_ASTRIPE = _AROWS // _NS


def _extract_i32(vec, i):
    return jnp.sum(jnp.where(_iota16() == i, vec, 0))


def _aggregate_sc(xlp, src, dst, lo_flat, mv, invd, qs, heads, e2p):
    dp = heads * 32
    dmsg = 96 if heads > 1 else 32
    mesh = plsc.VectorSubcoreMesh(core_axis_name="c", subcore_axis_name="s")

    @functools.partial(
        pl.kernel, mesh=mesh, compiler_params=_sc_params(),
        out_type=jax.ShapeDtypeStruct((_NQ, _AROWS, dmsg), jnp.float32),
        scratch_types=[
            pltpu.VMEM((_B,), jnp.int32),
            pltpu.VMEM((_B,), jnp.int32),
            pltpu.VMEM((_B,), jnp.int32),
            pltpu.VMEM((_B, dp), jnp.float32),
            pltpu.VMEM((_B, dmsg), jnp.float32),
            pltpu.VMEM((heads * _B,), jnp.float32),
            pltpu.VMEM((_B, 16), jnp.float32),
            pltpu.VMEM((16,), jnp.float32),
            pltpu.VMEM((16,), jnp.int32),
            pltpu.VMEM_SHARED((_AROWS, dmsg), jnp.float32),
            pltpu.SemaphoreType.DMA,
            pltpu.SemaphoreType.DMA,
        ],
    )
    def kern(xl_hbm, src_hbm, dst_hbm, lo_hbm, mv_hbm, inv_hbm, qs_hbm,
             zer_hbm, out_hbm, srcv, dstv, dloc, xlr, msg, lhv, invr, mvv,
             qsv, acc_sp, sem1, sem2):
        core = lax.axis_index("c")
        sid = lax.axis_index("s")
        pltpu.sync_copy(mv_hbm, mvv)
        pltpu.sync_copy(qs_hbm, qsv)
        pltpu.sync_copy(zer_hbm, msg)

        for j in range(_NQ // 2):
            q = core * (_NQ // 2) + j
            qsvv = qsv[...]
            qlo = _extract_i32(qsvv, q)
            qhi = _extract_i32(qsvv, q + 1)
            qbase = q * _QN
            per_t = (qhi - qlo + _NS - 1) // _NS
            s_k = qlo + sid * per_t
            e_k = jnp.minimum(s_k + per_t, qhi)
            s8 = (s_k // 8) * 8
            nch = jnp.maximum((e_k - s8 + _B - 1) // _B, 0)

            @pl.loop(0, _ASTRIPE // _B)
            def _za(i):
                pltpu.sync_copy(msg, acc_sp.at[pl.ds(sid * _ASTRIPE + i * _B,
                                                     _B)])

            rem = _ASTRIPE % _B
            if rem:
                pltpu.sync_copy(msg.at[pl.ds(0, rem)],
                                acc_sp.at[pl.ds(sid * _ASTRIPE
                                                + (_ASTRIPE // _B) * _B, rem)])
            plsc.subcore_barrier()

            def _chunk(ci, carry):
                base = s8 + ci * _B
                pltpu.sync_copy(src_hbm.at[pl.ds(base, _B)], srcv)
                pltpu.sync_copy(dst_hbm.at[pl.ds(base, _B)], dstv)
                cp1 = pltpu.async_copy(xl_hbm.at[srcv], xlr, sem1)
                cp2 = pltpu.async_copy(inv_hbm.at[dstv], invr, sem2)
                for h in range(heads):
                    pltpu.sync_copy(lo_hbm.at[pl.ds(h * e2p + base, _B)],
                                    lhv.at[pl.ds(h * _B, _B)])
                cp1.wait()
                cp2.wait()

                @plsc.parallel_loop(0, _B // 16, unroll=2)
                def _group(g):
                    ev = _iota16() + g * 16
                    eg = base + ev
                    inq = (eg >= s_k) & (eg < e_k)
                    dv = dstv[pl.ds(g * 16, 16)]
                    dloc[pl.ds(g * 16, 16)] = jnp.where(inq, dv - qbase, _QN)
                    for h in range(heads):
                        l = lhv[pl.ds(h * _B + g * 16, 16)]
                        ex = jnp.exp(l - mvv[...])
                        iv = plsc.load_gather(
                            invr, [ev, jnp.full((16,), h, jnp.int32)])
                        alpha = ex * iv
                        for c in range(_HID):
                            xv = plsc.load_gather(
                                xlr, [ev, jnp.full((16,), h * 32 + c,
                                                   jnp.int32)])
                            plsc.store_scatter(
                                msg, [ev, jnp.full((16,), h * _HID + c,
                                                   jnp.int32)],
                                xv * alpha)

                pltpu.sync_copy(msg, acc_sp.at[dloc], add=True)
                return carry

            lax.fori_loop(0, nch, _chunk, 0)
            plsc.subcore_barrier()

            @pl.loop(0, _ASTRIPE // _B)
            def _fl(i):
                pltpu.sync_copy(acc_sp.at[pl.ds(sid * _ASTRIPE + i * _B, _B)],
                                out_hbm.at[q, pl.ds(sid * _ASTRIPE + i * _B,
                                                    _B)])

            if rem:
                pltpu.sync_copy(
                    acc_sp.at[pl.ds(sid * _ASTRIPE + (_ASTRIPE // _B) * _B,
                                    rem)],
                    out_hbm.at[q, pl.ds(sid * _ASTRIPE + (_ASTRIPE // _B) * _B,
                                        rem)])

            if j < _NQ // 2 - 1:
                pltpu.sync_copy(zer_hbm, msg)

    return kern(xlp, src, dst, lo_flat, mv, invd, qs,
                jnp.zeros((_B, dmsg), jnp.float32))


_PB = 320
_PROWS = 528
_PSTRIPE = _PROWS // _NS


def _pool_sc(h_pad, batch_pad):
    per_tile = _NP // _NW
    n_chunks = per_tile // _PB
    mesh = plsc.VectorSubcoreMesh(core_axis_name="c", subcore_axis_name="s")

    @functools.partial(
        pl.kernel, mesh=mesh, compiler_params=_sc_params(),
        out_type=[jax.ShapeDtypeStruct((_NC, _PROWS, 32), jnp.float32),
                  jax.ShapeDtypeStruct((_NC, _PROWS, 32), jnp.float32)],
        scratch_types=[
            pltpu.VMEM((_PB,), jnp.int32),
            pltpu.VMEM((_PB, 32), jnp.float32),
            pltpu.VMEM((_PB, 32), jnp.float32),
            pltpu.VMEM_SHARED((_PROWS, 32), jnp.float32),
            pltpu.VMEM_SHARED((_PROWS, 32), jnp.float32),
        ],
    )
    def kern(h_hbm, b_hbm, ones_hbm, zer_hbm, sum_hbm, cnt_hbm, bv, hrows,
             ones, sum_sp, cnt_sp):
        core = lax.axis_index("c")
        sid = lax.axis_index("s")
        wid = sid * _NC + core
        tbase = wid * per_tile
        pltpu.sync_copy(ones_hbm, ones)
        pltpu.sync_copy(zer_hbm, hrows)

        pltpu.sync_copy(hrows.at[pl.ds(0, _PSTRIPE)],
                        sum_sp.at[pl.ds(sid * _PSTRIPE, _PSTRIPE)])
        pltpu.sync_copy(hrows.at[pl.ds(0, _PSTRIPE)],
                        cnt_sp.at[pl.ds(sid * _PSTRIPE, _PSTRIPE)])
        plsc.subcore_barrier()

        @pl.loop(0, n_chunks)
        def _chunk(ci):
            base = tbase + ci * _PB
            pltpu.sync_copy(b_hbm.at[pl.ds(base, _PB)], bv)
            pltpu.sync_copy(h_hbm.at[pl.ds(base, _PB)], hrows)
            pltpu.sync_copy(hrows, sum_sp.at[bv], add=True)
            pltpu.sync_copy(ones, cnt_sp.at[bv], add=True)

        plsc.subcore_barrier()
        pltpu.sync_copy(sum_sp.at[pl.ds(sid * _PSTRIPE, _PSTRIPE)],
                        sum_hbm.at[core, pl.ds(sid * _PSTRIPE, _PSTRIPE)])
        pltpu.sync_copy(cnt_sp.at[pl.ds(sid * _PSTRIPE, _PSTRIPE)],
                        cnt_hbm.at[core, pl.ds(sid * _PSTRIPE, _PSTRIPE)])

    return kern(h_pad, batch_pad, jnp.ones((_PB, 32), jnp.float32),
                jnp.zeros((_PB, 32), jnp.float32))


def _gatv2_layer(h_pad, src, dst, qs, wl, wr, att, bias, heads, e2, e2p):
    wlp = _pad_w(wl, heads)
    wrp = _pad_w(wr, heads)
    xlp, xrp = _proj(h_pad, wlp, wrp)
    att_rep = jnp.repeat(att.reshape(-1), 16)

    lo_flat, mx = _edge_logits_sc(xlp, xrp, src, dst, att_rep, heads, e2p)
    m = jnp.max(mx)
    mv = jnp.full((16,), m, jnp.float32)
    den2 = _den_sc(lo_flat, dst, mv, heads, e2p)
    invd = 1.0 / (den2[0] + den2[1] + 1e-16)
    out4 = _aggregate_sc(xlp, src, dst, lo_flat, mv, invd, qs, heads, e2p)
    d = heads * _HID
    out = out4[:, :_QN, :d].reshape(_NQ * _QN, d)
    return out + bias


def kernel(x, edge_index, batch, Wl1, Wr1, att1, b1, g1, be1, Wl2, Wr2, att2,
           b2, g2, be2, Wl3, Wr3, att3, b3, g3, be3):
    n = 50000
    e = edge_index.shape[1]
    e2 = e + n
    e2p = ((e2 + _NW * _B - 1) // (_NW * _B)) * (_NW * _B)

    loop = jnp.arange(n, dtype=jnp.int32)
    src0 = jnp.concatenate([edge_index[0], loop,
                            jnp.zeros((e2p - e2,), jnp.int32)])
    dst0 = jnp.concatenate([edge_index[1], loop,
                            jnp.full((e2p - e2,), n, jnp.int32)])
    order = jnp.argsort(dst0[:e2])
    src = jnp.concatenate([src0[order], src0[e2:]])
    dst = jnp.concatenate([dst0[order], dst0[e2:]])
    qs = jnp.searchsorted(
        dst, jnp.arange(0, (_NQ + 1) * _QN, _QN, dtype=jnp.int32))
    qs = jnp.concatenate([qs.astype(jnp.int32),
                          jnp.zeros((16 - _NQ - 1,), jnp.int32)])

    def pad_rows(h):
        return jnp.pad(h, ((0, _NP - h.shape[0]), (0, 0)))

    h = pad_rows(x.astype(jnp.float32))
    h = jnp.pad(h, ((0, 0), (0, 2)))
    h = _gatv2_layer(h, src, dst, qs, jnp.pad(Wl1, ((0, 2), (0, 0))),
                     jnp.pad(Wr1, ((0, 2), (0, 0))), att1, b1, _H, e2, e2p)
    h = _bn_elu(pad_rows(h), n, g1, be1)
    h = _gatv2_layer(h, src, dst, qs, Wl2, Wr2, att2, b2, _H, e2, e2p)
    h = _bn_elu(pad_rows(h), n, g2, be2)
    h = _gatv2_layer(h, src, dst, qs, Wl3, Wr3, att3, b3, 1, e2, e2p)
    h = _bn_elu(pad_rows(h), n, g3, be3)

    node_emb = h[:n]
    h32 = jnp.pad(h, ((0, 0), (0, 8)))
    batch_pad = jnp.concatenate([batch.astype(jnp.int32),
                                 jnp.full((_NP - n,), _G, jnp.int32)])
    sums2, cnts2 = _pool_sc(h32, batch_pad)
    sums = (sums2[0] + sums2[1])[:_G, :_HID]
    cnt = (cnts2[0] + cnts2[1])[:_G, :1]
    graph_emb = sums / jnp.maximum(cnt, 1.0)
    return (graph_emb, node_emb)


# speedup vs baseline: 14.0155x; 1.0174x over previous
import dataclasses
import functools

import jax
import jax.numpy as jnp
from jax import lax
from jax.experimental import pallas as pl
from jax.experimental.pallas import tpu as pltpu
from jax.experimental.pallas import tpu_sc as plsc

_H = 4
_HID = 24
_G = 512
_NP = 51200
_RBLK = 6400
_NC = 2
_NS = 16
_NW = _NC * _NS
_B = 256


def _sc_params():
    return dataclasses.replace(pltpu.CompilerParams(),
                               needs_layout_passes=False,
                               use_tc_tiling_on_sc=False)


def _proj_kernel(x_ref, wl_ref, wr_ref, xl_ref, xr_ref):
    x = x_ref[...]
    xl_ref[...] = jnp.dot(x, wl_ref[...], preferred_element_type=jnp.float32)
    xr_ref[...] = jnp.dot(x, wr_ref[...], preferred_element_type=jnp.float32)


def _proj(x_pad, wlp, wrp):
    k, dp = wlp.shape
    grid = _NP // _RBLK
    return pl.pallas_call(
        _proj_kernel,
        grid=(grid,),
        in_specs=[pl.BlockSpec((_RBLK, k), lambda i: (i, 0)),
                  pl.BlockSpec((k, dp), lambda i: (0, 0)),
                  pl.BlockSpec((k, dp), lambda i: (0, 0))],
        out_specs=[pl.BlockSpec((_RBLK, dp), lambda i: (i, 0)),
                   pl.BlockSpec((_RBLK, dp), lambda i: (i, 0))],
        out_shape=[jax.ShapeDtypeStruct((_NP, dp), jnp.float32),
                   jax.ShapeDtypeStruct((_NP, dp), jnp.float32)],
    )(x_pad, wlp, wrp)


def _pad_w(w, heads):
    k = w.shape[0]
    w = w.reshape(k, heads, _HID)
    w = jnp.pad(w, ((0, 0), (0, 0), (0, 32 - _HID)))
    return w.reshape(k, heads * 32)


_BN_BLK = _RBLK


def _bn_stats_kernel(h_ref, s_ref, q_ref):
    i = pl.program_id(0)

    @pl.when(i == 0)
    def _():
        s_ref[...] = jnp.zeros_like(s_ref)
        q_ref[...] = jnp.zeros_like(q_ref)

    h = h_ref[...]
    s_ref[...] += jnp.sum(h, axis=0, keepdims=True)
    q_ref[...] += jnp.sum(h * h, axis=0, keepdims=True)


def _bn_apply_kernel(h_ref, mu_ref, isd_ref, g_ref, be_ref, o_ref):
    y = g_ref[...] * (h_ref[...] - mu_ref[...]) * isd_ref[...] + be_ref[...]
    o_ref[...] = jnp.where(y > 0, y, jnp.exp(jnp.minimum(y, 0.0)) - 1.0)


def _bn_elu(h_pad, n_real, gamma, beta):
    np_, c = h_pad.shape
    nb = np_ // _BN_BLK
    s, q = pl.pallas_call(
        _bn_stats_kernel,
        grid=(nb,),
        in_specs=[pl.BlockSpec((_BN_BLK, c), lambda i: (i, 0))],
        out_specs=[pl.BlockSpec((1, c), lambda i: (0, 0)),
                   pl.BlockSpec((1, c), lambda i: (0, 0))],
        out_shape=[jax.ShapeDtypeStruct((1, c), jnp.float32),
                   jax.ShapeDtypeStruct((1, c), jnp.float32)],
    )(h_pad)
    mu = s / n_real
    var = q / n_real - mu * mu
    isd = 1.0 / jnp.sqrt(var + 1e-5)
    return pl.pallas_call(
        _bn_apply_kernel,
        grid=(nb,),
        in_specs=[pl.BlockSpec((_BN_BLK, c), lambda i: (i, 0)),
                  pl.BlockSpec((1, c), lambda i: (0, 0)),
                  pl.BlockSpec((1, c), lambda i: (0, 0)),
                  pl.BlockSpec((1, c), lambda i: (0, 0)),
                  pl.BlockSpec((1, c), lambda i: (0, 0))],
        out_specs=pl.BlockSpec((_BN_BLK, c), lambda i: (i, 0)),
        out_shape=jax.ShapeDtypeStruct((np_, c), jnp.float32),
    )(h_pad, mu, isd, gamma.reshape(1, c), beta.reshape(1, c))


def _iota16():
    return lax.broadcasted_iota(jnp.int32, (16,), 0)


def _edge_logits_sc(xlp, xrp, src, dst, att_rep, heads, e2p):
    dp = heads * 32
    bb = 128
    per_tile = e2p // _NW
    n_chunks = per_tile // bb
    mesh = plsc.VectorSubcoreMesh(core_axis_name="c", subcore_axis_name="s")

    @functools.partial(
        pl.kernel, mesh=mesh, compiler_params=_sc_params(),
        out_type=[jax.ShapeDtypeStruct((heads * e2p,), jnp.float32),
                  jax.ShapeDtypeStruct((_NW, 16), jnp.float32)],
        scratch_types=[
            pltpu.VMEM((per_tile,), jnp.int32),
            pltpu.VMEM((per_tile,), jnp.int32),
            pltpu.VMEM((bb, dp), jnp.float32),
            pltpu.VMEM((bb, dp), jnp.float32),
            pltpu.VMEM((bb, dp), jnp.float32),
            pltpu.VMEM((bb, dp), jnp.float32),
            pltpu.VMEM((heads * bb,), jnp.float32),
            pltpu.VMEM((heads * bb,), jnp.float32),
            pltpu.VMEM((heads * _HID * 16,), jnp.float32),
            pltpu.VMEM((16,), jnp.float32),
            pltpu.SemaphoreType.DMA,
            pltpu.SemaphoreType.DMA,
            pltpu.SemaphoreType.DMA,
            pltpu.SemaphoreType.DMA,
            pltpu.SemaphoreType.DMA,
            pltpu.SemaphoreType.DMA,
        ],
    )
    def kern(xl_hbm, xr_hbm, src_hbm, dst_hbm, att_hbm, lo_hbm, mx_hbm,
             src_all, dst_all, xlr0, xrr0, xlr1, xrr1, lch0, lch1, attv, mxv,
             sl0, sr0, sl1, sr1, so0, so1):
        wid = lax.axis_index("s") * _NC + lax.axis_index("c")
        tbase = wid * per_tile
        pltpu.sync_copy(att_hbm, attv)
        pltpu.sync_copy(src_hbm.at[pl.ds(tbase, per_tile)], src_all)
        pltpu.sync_copy(dst_hbm.at[pl.ds(tbase, per_tile)], dst_all)
        mxv[...] = jnp.full((16,), -3e38, jnp.float32)

        def compute(ci, xlr, xrr, lchunk):
            @plsc.parallel_loop(0, bb // 16, unroll=2,
                                carry=jnp.full((16,), -3e38, jnp.float32))
            def _group(g, mxc):
                ev = _iota16() + g * 16
                for h in range(heads):
                    acc = jnp.zeros((16,), jnp.float32)
                    for c in range(_HID):
                        cv = jnp.full((16,), h * 32 + c, jnp.int32)
                        a = plsc.load_gather(xlr, [ev, cv])
                        b = plsc.load_gather(xrr, [ev, cv])
                        z = a + b
                        lr = jnp.maximum(z, 0.0) + 0.2 * jnp.minimum(z, 0.0)
                        av = attv[pl.ds((h * _HID + c) * 16, 16)]
                        acc = acc + lr * av
                    lchunk[pl.ds(h * bb + g * 16, 16)] = acc
                    mxc = jnp.maximum(mxc, acc)
                return mxc

            mxv[...] = jnp.maximum(mxv[...], _group)

        def store_out(ci, lchunk, sem):
            base = tbase + ci * bb
            return [pltpu.async_copy(lchunk.at[pl.ds(h * bb, bb)],
                                     lo_hbm.at[pl.ds(h * e2p + base, bb)],
                                     sem)
                    for h in range(heads)]

        @pl.loop(0, n_chunks // 2)
        def _pair(i):
            a = 2 * i
            b = a + 1
            ga1 = pltpu.async_copy(
                xl_hbm.at[src_all.at[pl.ds(a * bb, bb)]], xlr0, sl0)
            ga2 = pltpu.async_copy(
                xr_hbm.at[dst_all.at[pl.ds(a * bb, bb)]], xrr0, sr0)
            gb1 = pltpu.async_copy(
                xl_hbm.at[src_all.at[pl.ds(b * bb, bb)]], xlr1, sl1)
            gb2 = pltpu.async_copy(
                xr_hbm.at[dst_all.at[pl.ds(b * bb, bb)]], xrr1, sr1)
            ga1.wait()
            ga2.wait()
            compute(a, xlr0, xrr0, lch0)
            oa = store_out(a, lch0, so0)
            gb1.wait()
            gb2.wait()
            compute(b, xlr1, xrr1, lch1)
            ob = store_out(b, lch1, so1)
            for cp in oa:
                cp.wait()
            for cp in ob:
                cp.wait()

        pltpu.sync_copy(mxv, mx_hbm.at[wid])

    return kern(xlp, xrp, src, dst, att_rep)


_DN = 50176
_DSTRIPE = _DN // _NS


def _den_sc(lo_flat, dst, mv, heads, e2p):
    per_tile = e2p // _NW
    n_chunks = per_tile // _B
    mesh = plsc.VectorSubcoreMesh(core_axis_name="c", subcore_axis_name="s")

    @functools.partial(
        pl.kernel, mesh=mesh, compiler_params=_sc_params(),
        out_type=jax.ShapeDtypeStruct((_NC, _DN, 16), jnp.float32),
        scratch_types=[
            pltpu.VMEM((_B,), jnp.int32),
            pltpu.VMEM((heads * _B,), jnp.float32),
            pltpu.VMEM((_B, 16), jnp.float32),
            pltpu.VMEM((16,), jnp.float32),
            pltpu.VMEM_SHARED((_DN, 16), jnp.float32),
        ],
    )
    def kern(lo_hbm, dst_hbm, mv_hbm, zer_hbm, den_hbm, dstv, lhv, exr, mvv,
             den_sp):
        core = lax.axis_index("c")
        sid = lax.axis_index("s")
        wid = sid * _NC + core
        tbase = wid * per_tile
        pltpu.sync_copy(mv_hbm, mvv)
        pltpu.sync_copy(zer_hbm, exr)

        @pl.loop(0, _DSTRIPE // _B)
        def _zs(i):
            pltpu.sync_copy(exr, den_sp.at[pl.ds(sid * _DSTRIPE + i * _B, _B)])

        rem = _DSTRIPE % _B
        if rem:
            pltpu.sync_copy(exr.at[pl.ds(0, rem)],
                            den_sp.at[pl.ds(sid * _DSTRIPE
                                            + (_DSTRIPE // _B) * _B, rem)])
        plsc.subcore_barrier()

        @pl.loop(0, n_chunks)
        def _chunk(ci):
            base = tbase + ci * _B
            pltpu.sync_copy(dst_hbm.at[pl.ds(base, _B)], dstv)
            for h in range(heads):
                pltpu.sync_copy(lo_hbm.at[pl.ds(h * e2p + base, _B)],
                                lhv.at[pl.ds(h * _B, _B)])

            @plsc.parallel_loop(0, _B // 16, unroll=2)
            def _group(g):
                ev = _iota16() + g * 16
                for h in range(heads):
                    l = lhv[pl.ds(h * _B + g * 16, 16)]
                    ex = jnp.exp(l - mvv[...])
                    plsc.store_scatter(exr, [ev, jnp.full((16,), h, jnp.int32)],
                                       ex)
            pltpu.sync_copy(exr, den_sp.at[dstv], add=True)

        plsc.subcore_barrier()
        pltpu.sync_copy(den_sp.at[pl.ds(sid * _DSTRIPE, _DSTRIPE)],
                        den_hbm.at[core, pl.ds(sid * _DSTRIPE, _DSTRIPE)])

    return kern(lo_flat, dst, mv, jnp.zeros((_B, 16), jnp.float32))


_QN = 6250
_NQ = 8
_AROWS = 6272
_ASTRIPE = _AROWS // _NS


def _extract_i32(vec, i):
    return jnp.sum(jnp.where(_iota16() == i, vec, 0))


def _aggregate_sc(xlp, src, dst, lo_flat, mv, invd, qs, heads, e2p):
    dp = heads * 32
    dmsg = 96 if heads > 1 else 32
    mesh = plsc.VectorSubcoreMesh(core_axis_name="c", subcore_axis_name="s")

    @functools.partial(
        pl.kernel, mesh=mesh, compiler_params=_sc_params(),
        out_type=jax.ShapeDtypeStruct((_NQ, _AROWS, dmsg), jnp.float32),
        scratch_types=[
            pltpu.VMEM((_B,), jnp.int32),
            pltpu.VMEM((_B,), jnp.int32),
            pltpu.VMEM((_B,), jnp.int32),
            pltpu.VMEM((_B, dp), jnp.float32),
            pltpu.VMEM((_B, dmsg), jnp.float32),
            pltpu.VMEM((heads * _B,), jnp.float32),
            pltpu.VMEM((_B, 16), jnp.float32),
            pltpu.VMEM((16,), jnp.float32),
            pltpu.VMEM((16,), jnp.int32),
            pltpu.VMEM_SHARED((_AROWS, dmsg), jnp.float32),
            pltpu.SemaphoreType.DMA,
            pltpu.SemaphoreType.DMA,
        ],
    )
    def kern(xl_hbm, src_hbm, dst_hbm, lo_hbm, mv_hbm, inv_hbm, qs_hbm,
             zer_hbm, out_hbm, srcv, dstv, dloc, xlr, msg, lhv, invr, mvv,
             qsv, acc_sp, sem1, sem2):
        core = lax.axis_index("c")
        sid = lax.axis_index("s")
        pltpu.sync_copy(mv_hbm, mvv)
        pltpu.sync_copy(qs_hbm, qsv)
        pltpu.sync_copy(zer_hbm, msg)

        for j in range(_NQ // 2):
            q = core * (_NQ // 2) + j
            qsvv = qsv[...]
            qlo = _extract_i32(qsvv, q)
            qhi = _extract_i32(qsvv, q + 1)
            qbase = q * _QN
            per_t = (qhi - qlo + _NS - 1) // _NS
            s_k = qlo + sid * per_t
            e_k = jnp.minimum(s_k + per_t, qhi)
            s8 = (s_k // 8) * 8
            nch = jnp.maximum((e_k - s8 + _B - 1) // _B, 0)

            @pl.loop(0, _ASTRIPE // _B)
            def _za(i):
                pltpu.sync_copy(msg, acc_sp.at[pl.ds(sid * _ASTRIPE + i * _B,
                                                     _B)])

            rem = _ASTRIPE % _B
            if rem:
                pltpu.sync_copy(msg.at[pl.ds(0, rem)],
                                acc_sp.at[pl.ds(sid * _ASTRIPE
                                                + (_ASTRIPE // _B) * _B, rem)])
            plsc.subcore_barrier()

            def _chunk(ci, carry):
                base = s8 + ci * _B
                pltpu.sync_copy(src_hbm.at[pl.ds(base, _B)], srcv)
                pltpu.sync_copy(dst_hbm.at[pl.ds(base, _B)], dstv)
                cp1 = pltpu.async_copy(xl_hbm.at[srcv], xlr, sem1)
                cp2 = pltpu.async_copy(inv_hbm.at[dstv], invr, sem2)
                for h in range(heads):
                    pltpu.sync_copy(lo_hbm.at[pl.ds(h * e2p + base, _B)],
                                    lhv.at[pl.ds(h * _B, _B)])
                cp1.wait()
                cp2.wait()

                @plsc.parallel_loop(0, _B // 16, unroll=2)
                def _group(g):
                    ev = _iota16() + g * 16
                    eg = base + ev
                    inq = (eg >= s_k) & (eg < e_k)
                    dv = dstv[pl.ds(g * 16, 16)]
                    dloc[pl.ds(g * 16, 16)] = jnp.where(inq, dv - qbase, _QN)
                    for h in range(heads):
                        l = lhv[pl.ds(h * _B + g * 16, 16)]
                        ex = jnp.exp(l - mvv[...])
                        iv = plsc.load_gather(
                            invr, [ev, jnp.full((16,), h, jnp.int32)])
                        alpha = ex * iv
                        for c in range(_HID):
                            xv = plsc.load_gather(
                                xlr, [ev, jnp.full((16,), h * 32 + c,
                                                   jnp.int32)])
                            plsc.store_scatter(
                                msg, [ev, jnp.full((16,), h * _HID + c,
                                                   jnp.int32)],
                                xv * alpha)

                pltpu.sync_copy(msg, acc_sp.at[dloc], add=True)
                return carry

            lax.fori_loop(0, nch, _chunk, 0)
            plsc.subcore_barrier()

            @pl.loop(0, _ASTRIPE // _B)
            def _fl(i):
                pltpu.sync_copy(acc_sp.at[pl.ds(sid * _ASTRIPE + i * _B, _B)],
                                out_hbm.at[q, pl.ds(sid * _ASTRIPE + i * _B,
                                                    _B)])

            if rem:
                pltpu.sync_copy(
                    acc_sp.at[pl.ds(sid * _ASTRIPE + (_ASTRIPE // _B) * _B,
                                    rem)],
                    out_hbm.at[q, pl.ds(sid * _ASTRIPE + (_ASTRIPE // _B) * _B,
                                        rem)])

            if j < _NQ // 2 - 1:
                pltpu.sync_copy(zer_hbm, msg)

    return kern(xlp, src, dst, lo_flat, mv, invd, qs,
                jnp.zeros((_B, dmsg), jnp.float32))


_PB = 320
_PROWS = 528
_PSTRIPE = _PROWS // _NS


def _pool_sc(h_pad, batch_pad):
    per_tile = _NP // _NW
    n_chunks = per_tile // _PB
    mesh = plsc.VectorSubcoreMesh(core_axis_name="c", subcore_axis_name="s")

    @functools.partial(
        pl.kernel, mesh=mesh, compiler_params=_sc_params(),
        out_type=[jax.ShapeDtypeStruct((_NC, _PROWS, 32), jnp.float32),
                  jax.ShapeDtypeStruct((_NC, _PROWS, 32), jnp.float32)],
        scratch_types=[
            pltpu.VMEM((_PB,), jnp.int32),
            pltpu.VMEM((_PB, 32), jnp.float32),
            pltpu.VMEM((_PB, 32), jnp.float32),
            pltpu.VMEM_SHARED((_PROWS, 32), jnp.float32),
            pltpu.VMEM_SHARED((_PROWS, 32), jnp.float32),
        ],
    )
    def kern(h_hbm, b_hbm, ones_hbm, zer_hbm, sum_hbm, cnt_hbm, bv, hrows,
             ones, sum_sp, cnt_sp):
        core = lax.axis_index("c")
        sid = lax.axis_index("s")
        wid = sid * _NC + core
        tbase = wid * per_tile
        pltpu.sync_copy(ones_hbm, ones)
        pltpu.sync_copy(zer_hbm, hrows)

        pltpu.sync_copy(hrows.at[pl.ds(0, _PSTRIPE)],
                        sum_sp.at[pl.ds(sid * _PSTRIPE, _PSTRIPE)])
        pltpu.sync_copy(hrows.at[pl.ds(0, _PSTRIPE)],
                        cnt_sp.at[pl.ds(sid * _PSTRIPE, _PSTRIPE)])
        plsc.subcore_barrier()

        @pl.loop(0, n_chunks)
        def _chunk(ci):
            base = tbase + ci * _PB
            pltpu.sync_copy(b_hbm.at[pl.ds(base, _PB)], bv)
            pltpu.sync_copy(h_hbm.at[pl.ds(base, _PB)], hrows)
            pltpu.sync_copy(hrows, sum_sp.at[bv], add=True)
            pltpu.sync_copy(ones, cnt_sp.at[bv], add=True)

        plsc.subcore_barrier()
        pltpu.sync_copy(sum_sp.at[pl.ds(sid * _PSTRIPE, _PSTRIPE)],
                        sum_hbm.at[core, pl.ds(sid * _PSTRIPE, _PSTRIPE)])
        pltpu.sync_copy(cnt_sp.at[pl.ds(sid * _PSTRIPE, _PSTRIPE)],
                        cnt_hbm.at[core, pl.ds(sid * _PSTRIPE, _PSTRIPE)])

    return kern(h_pad, batch_pad, jnp.ones((_PB, 32), jnp.float32),
                jnp.zeros((_PB, 32), jnp.float32))


def _gatv2_layer(h_pad, src, dst, qs, wl, wr, att, bias, heads, e2, e2p):
    wlp = _pad_w(wl, heads)
    wrp = _pad_w(wr, heads)
    xlp, xrp = _proj(h_pad, wlp, wrp)
    att_rep = jnp.repeat(att.reshape(-1), 16)

    lo_flat, mx = _edge_logits_sc(xlp, xrp, src, dst, att_rep, heads, e2p)
    m = jnp.max(mx)
    mv = jnp.full((16,), m, jnp.float32)
    den2 = _den_sc(lo_flat, dst, mv, heads, e2p)
    invd = 1.0 / (den2[0] + den2[1] + 1e-16)
    out4 = _aggregate_sc(xlp, src, dst, lo_flat, mv, invd, qs, heads, e2p)
    d = heads * _HID
    out = out4[:, :_QN, :d].reshape(_NQ * _QN, d)
    return out + bias


def kernel(x, edge_index, batch, Wl1, Wr1, att1, b1, g1, be1, Wl2, Wr2, att2,
           b2, g2, be2, Wl3, Wr3, att3, b3, g3, be3):
    n = 50000
    e = edge_index.shape[1]
    e2 = e + n
    e2p = ((e2 + _NW * _B - 1) // (_NW * _B)) * (_NW * _B)

    loop = jnp.arange(n, dtype=jnp.int32)
    src0 = jnp.concatenate([edge_index[0], loop,
                            jnp.zeros((e2p - e2,), jnp.int32)])
    dst0 = jnp.concatenate([edge_index[1], loop,
                            jnp.full((e2p - e2,), n, jnp.int32)])
    order = jnp.argsort(dst0[:e2])
    src = jnp.concatenate([src0[order], src0[e2:]])
    dst = jnp.concatenate([dst0[order], dst0[e2:]])
    qs = jnp.searchsorted(
        dst, jnp.arange(0, (_NQ + 1) * _QN, _QN, dtype=jnp.int32))
    qs = jnp.concatenate([qs.astype(jnp.int32),
                          jnp.zeros((16 - _NQ - 1,), jnp.int32)])

    def pad_rows(h):
        return jnp.pad(h, ((0, _NP - h.shape[0]), (0, 0)))

    h = pad_rows(x.astype(jnp.float32))
    h = jnp.pad(h, ((0, 0), (0, 2)))
    h = _gatv2_layer(h, src, dst, qs, jnp.pad(Wl1, ((0, 2), (0, 0))),
                     jnp.pad(Wr1, ((0, 2), (0, 0))), att1, b1, _H, e2, e2p)
    h = _bn_elu(pad_rows(h), n, g1, be1)
    h = _gatv2_layer(h, src, dst, qs, Wl2, Wr2, att2, b2, _H, e2, e2p)
    h = _bn_elu(pad_rows(h), n, g2, be2)
    h = _gatv2_layer(h, src, dst, qs, Wl3, Wr3, att3, b3, 1, e2, e2p)
    h = _bn_elu(pad_rows(h), n, g3, be3)

    node_emb = h[:n]
    h32 = jnp.pad(h, ((0, 0), (0, 8)))
    batch_pad = jnp.concatenate([batch.astype(jnp.int32),
                                 jnp.full((_NP - n,), _G, jnp.int32)])
    sums2, cnts2 = _pool_sc(h32, batch_pad)
    sums = (sums2[0] + sums2[1])[:_G, :_HID]
    cnt = (cnts2[0] + cnts2[1])[:_G, :1]
    graph_emb = sums / jnp.maximum(cnt, 1.0)
    return (graph_emb, node_emb)


# speedup vs baseline: 14.7187x; 1.0502x over previous
import dataclasses
import functools

import jax
import jax.numpy as jnp
from jax import lax
from jax.experimental import pallas as pl
from jax.experimental.pallas import tpu as pltpu
from jax.experimental.pallas import tpu_sc as plsc

_H = 4
_HID = 24
_G = 512
_NP = 51200
_RBLK = 6400
_NC = 2
_NS = 16
_NW = _NC * _NS
_B = 256


def _sc_params():
    return dataclasses.replace(pltpu.CompilerParams(),
                               needs_layout_passes=False,
                               use_tc_tiling_on_sc=False)


def _proj_kernel(x_ref, wl_ref, wr_ref, xl_ref, xr_ref):
    x = x_ref[...]
    xl_ref[...] = jnp.dot(x, wl_ref[...], preferred_element_type=jnp.float32)
    xr_ref[...] = jnp.dot(x, wr_ref[...], preferred_element_type=jnp.float32)


def _proj(x_pad, wlp, wrp):
    k, dp = wlp.shape
    grid = _NP // _RBLK
    return pl.pallas_call(
        _proj_kernel,
        grid=(grid,),
        in_specs=[pl.BlockSpec((_RBLK, k), lambda i: (i, 0)),
                  pl.BlockSpec((k, dp), lambda i: (0, 0)),
                  pl.BlockSpec((k, dp), lambda i: (0, 0))],
        out_specs=[pl.BlockSpec((_RBLK, dp), lambda i: (i, 0)),
                   pl.BlockSpec((_RBLK, dp), lambda i: (i, 0))],
        out_shape=[jax.ShapeDtypeStruct((_NP, dp), jnp.float32),
                   jax.ShapeDtypeStruct((_NP, dp), jnp.float32)],
    )(x_pad, wlp, wrp)


def _pad_w(w, heads):
    k = w.shape[0]
    w = w.reshape(k, heads, _HID)
    w = jnp.pad(w, ((0, 0), (0, 0), (0, 32 - _HID)))
    return w.reshape(k, heads * 32)


_BN_BLK = _RBLK


def _bn_stats_kernel(h_ref, s_ref, q_ref):
    i = pl.program_id(0)

    @pl.when(i == 0)
    def _():
        s_ref[...] = jnp.zeros_like(s_ref)
        q_ref[...] = jnp.zeros_like(q_ref)

    h = h_ref[...]
    s_ref[...] += jnp.sum(h, axis=0, keepdims=True)
    q_ref[...] += jnp.sum(h * h, axis=0, keepdims=True)


def _bn_apply_kernel(h_ref, mu_ref, isd_ref, g_ref, be_ref, o_ref):
    y = g_ref[...] * (h_ref[...] - mu_ref[...]) * isd_ref[...] + be_ref[...]
    o_ref[...] = jnp.where(y > 0, y, jnp.exp(jnp.minimum(y, 0.0)) - 1.0)


def _bn_elu(h_pad, n_real, gamma, beta):
    np_, c = h_pad.shape
    nb = np_ // _BN_BLK
    s, q = pl.pallas_call(
        _bn_stats_kernel,
        grid=(nb,),
        in_specs=[pl.BlockSpec((_BN_BLK, c), lambda i: (i, 0))],
        out_specs=[pl.BlockSpec((1, c), lambda i: (0, 0)),
                   pl.BlockSpec((1, c), lambda i: (0, 0))],
        out_shape=[jax.ShapeDtypeStruct((1, c), jnp.float32),
                   jax.ShapeDtypeStruct((1, c), jnp.float32)],
    )(h_pad)
    mu = s / n_real
    var = q / n_real - mu * mu
    isd = 1.0 / jnp.sqrt(var + 1e-5)
    return pl.pallas_call(
        _bn_apply_kernel,
        grid=(nb,),
        in_specs=[pl.BlockSpec((_BN_BLK, c), lambda i: (i, 0)),
                  pl.BlockSpec((1, c), lambda i: (0, 0)),
                  pl.BlockSpec((1, c), lambda i: (0, 0)),
                  pl.BlockSpec((1, c), lambda i: (0, 0)),
                  pl.BlockSpec((1, c), lambda i: (0, 0))],
        out_specs=pl.BlockSpec((_BN_BLK, c), lambda i: (i, 0)),
        out_shape=jax.ShapeDtypeStruct((np_, c), jnp.float32),
    )(h_pad, mu, isd, gamma.reshape(1, c), beta.reshape(1, c))


def _iota16():
    return lax.broadcasted_iota(jnp.int32, (16,), 0)


def _edge_logits_sc(xlp, xrp, src, dst, att_rep, heads, e2p):
    dp = heads * 32
    bb = 128
    per_tile = e2p // _NW
    n_chunks = per_tile // bb
    mesh = plsc.VectorSubcoreMesh(core_axis_name="c", subcore_axis_name="s")

    @functools.partial(
        pl.kernel, mesh=mesh, compiler_params=_sc_params(),
        out_type=[jax.ShapeDtypeStruct((heads * e2p,), jnp.float32),
                  jax.ShapeDtypeStruct((_NW, 16), jnp.float32)],
        scratch_types=[
            pltpu.VMEM((per_tile,), jnp.int32),
            pltpu.VMEM((per_tile,), jnp.int32),
            pltpu.VMEM((bb, dp), jnp.float32),
            pltpu.VMEM((bb, dp), jnp.float32),
            pltpu.VMEM((bb, dp), jnp.float32),
            pltpu.VMEM((bb, dp), jnp.float32),
            pltpu.VMEM((heads * bb,), jnp.float32),
            pltpu.VMEM((heads * bb,), jnp.float32),
            pltpu.VMEM((heads * _HID * 16,), jnp.float32),
            pltpu.VMEM((16,), jnp.float32),
            pltpu.SemaphoreType.DMA,
            pltpu.SemaphoreType.DMA,
            pltpu.SemaphoreType.DMA,
            pltpu.SemaphoreType.DMA,
            pltpu.SemaphoreType.DMA,
            pltpu.SemaphoreType.DMA,
        ],
    )
    def kern(xl_hbm, xr_hbm, src_hbm, dst_hbm, att_hbm, lo_hbm, mx_hbm,
             src_all, dst_all, xlr0, xrr0, xlr1, xrr1, lch0, lch1, attv, mxv,
             sl0, sr0, sl1, sr1, so0, so1):
        wid = lax.axis_index("s") * _NC + lax.axis_index("c")
        tbase = wid * per_tile
        pltpu.sync_copy(att_hbm, attv)
        pltpu.sync_copy(src_hbm.at[pl.ds(tbase, per_tile)], src_all)
        pltpu.sync_copy(dst_hbm.at[pl.ds(tbase, per_tile)], dst_all)
        mxv[...] = jnp.full((16,), -3e38, jnp.float32)

        def compute(ci, xlr, xrr, lchunk):
            @plsc.parallel_loop(0, bb // 16, unroll=2,
                                carry=jnp.full((16,), -3e38, jnp.float32))
            def _group(g, mxc):
                ev = _iota16() + g * 16
                for h in range(heads):
                    acc = jnp.zeros((16,), jnp.float32)
                    for c in range(_HID):
                        cv = jnp.full((16,), h * 32 + c, jnp.int32)
                        a = plsc.load_gather(xlr, [ev, cv])
                        b = plsc.load_gather(xrr, [ev, cv])
                        z = a + b
                        lr = jnp.maximum(z, 0.0) + 0.2 * jnp.minimum(z, 0.0)
                        av = attv[pl.ds((h * _HID + c) * 16, 16)]
                        acc = acc + lr * av
                    lchunk[pl.ds(h * bb + g * 16, 16)] = acc
                    mxc = jnp.maximum(mxc, acc)
                return mxc

            mxv[...] = jnp.maximum(mxv[...], _group)

        def store_out(ci, lchunk, sem):
            base = tbase + ci * bb
            return [pltpu.async_copy(lchunk.at[pl.ds(h * bb, bb)],
                                     lo_hbm.at[pl.ds(h * e2p + base, bb)],
                                     sem)
                    for h in range(heads)]

        @pl.loop(0, n_chunks // 2)
        def _pair(i):
            a = 2 * i
            b = a + 1
            ga1 = pltpu.async_copy(
                xl_hbm.at[src_all.at[pl.ds(a * bb, bb)]], xlr0, sl0)
            ga2 = pltpu.async_copy(
                xr_hbm.at[dst_all.at[pl.ds(a * bb, bb)]], xrr0, sr0)
            gb1 = pltpu.async_copy(
                xl_hbm.at[src_all.at[pl.ds(b * bb, bb)]], xlr1, sl1)
            gb2 = pltpu.async_copy(
                xr_hbm.at[dst_all.at[pl.ds(b * bb, bb)]], xrr1, sr1)
            ga1.wait()
            ga2.wait()
            compute(a, xlr0, xrr0, lch0)
            oa = store_out(a, lch0, so0)
            gb1.wait()
            gb2.wait()
            compute(b, xlr1, xrr1, lch1)
            ob = store_out(b, lch1, so1)
            for cp in oa:
                cp.wait()
            for cp in ob:
                cp.wait()

        pltpu.sync_copy(mxv, mx_hbm.at[wid])

    return kern(xlp, xrp, src, dst, att_rep)


_DN = 50176
_DSTRIPE = _DN // _NS


def _den_sc(lo_flat, dst, mv, heads, e2p):
    per_tile = e2p // _NW
    n_chunks = per_tile // _B
    mesh = plsc.VectorSubcoreMesh(core_axis_name="c", subcore_axis_name="s")

    @functools.partial(
        pl.kernel, mesh=mesh, compiler_params=_sc_params(),
        out_type=jax.ShapeDtypeStruct((_NC, _DN, 16), jnp.float32),
        scratch_types=[
            pltpu.VMEM((_B,), jnp.int32),
            pltpu.VMEM((_B,), jnp.int32),
            pltpu.VMEM((heads * _B,), jnp.float32),
            pltpu.VMEM((heads * _B,), jnp.float32),
            pltpu.VMEM((_B, 16), jnp.float32),
            pltpu.VMEM((_B, 16), jnp.float32),
            pltpu.VMEM((16,), jnp.float32),
            pltpu.VMEM_SHARED((_DN, 16), jnp.float32),
            pltpu.SemaphoreType.DMA,
            pltpu.SemaphoreType.DMA,
            pltpu.SemaphoreType.DMA,
            pltpu.SemaphoreType.DMA,
        ],
    )
    def kern(lo_hbm, dst_hbm, mv_hbm, zer_hbm, den_hbm, dstv0, dstv1, lhv0,
             lhv1, exr0, exr1, mvv, den_sp, s0, s1, sc0, sc1):
        core = lax.axis_index("c")
        sid = lax.axis_index("s")
        wid = sid * _NC + core
        tbase = wid * per_tile
        pltpu.sync_copy(mv_hbm, mvv)
        pltpu.sync_copy(zer_hbm, exr0)
        pltpu.sync_copy(zer_hbm, exr1)

        @pl.loop(0, _DSTRIPE // _B)
        def _zs(i):
            pltpu.sync_copy(exr0, den_sp.at[pl.ds(sid * _DSTRIPE + i * _B,
                                                  _B)])

        rem = _DSTRIPE % _B
        if rem:
            pltpu.sync_copy(exr0.at[pl.ds(0, rem)],
                            den_sp.at[pl.ds(sid * _DSTRIPE
                                            + (_DSTRIPE // _B) * _B, rem)])
        plsc.subcore_barrier()

        def load(ci, dstv, lhv, sem):
            base = tbase + ci * _B
            cps = [pltpu.async_copy(dst_hbm.at[pl.ds(base, _B)], dstv, sem)]
            for h in range(heads):
                cps.append(pltpu.async_copy(
                    lo_hbm.at[pl.ds(h * e2p + base, _B)],
                    lhv.at[pl.ds(h * _B, _B)], sem))
            return cps

        def compute(lhv, exr):
            @plsc.parallel_loop(0, _B // 16, unroll=2)
            def _group(g):
                ev = _iota16() + g * 16
                for h in range(heads):
                    l = lhv[pl.ds(h * _B + g * 16, 16)]
                    ex = jnp.exp(l - mvv[...])
                    plsc.store_scatter(exr, [ev, jnp.full((16,), h, jnp.int32)],
                                       ex)

        @pl.loop(0, n_chunks // 2)
        def _pair(i):
            a = 2 * i
            la = load(a, dstv0, lhv0, s0)
            lb = load(a + 1, dstv1, lhv1, s1)
            for cp in la:
                cp.wait()
            compute(lhv0, exr0)
            sca = pltpu.async_copy(exr0, den_sp.at[dstv0], sc0, add=True)
            for cp in lb:
                cp.wait()
            compute(lhv1, exr1)
            scb = pltpu.async_copy(exr1, den_sp.at[dstv1], sc1, add=True)
            sca.wait()
            scb.wait()

        plsc.subcore_barrier()
        pltpu.sync_copy(den_sp.at[pl.ds(sid * _DSTRIPE, _DSTRIPE)],
                        den_hbm.at[core, pl.ds(sid * _DSTRIPE, _DSTRIPE)])

    return kern(lo_flat, dst, mv, jnp.zeros((_B, 16), jnp.float32))


_QN = 6250
_NQ = 8
_AROWS = 6272
_ASTRIPE = _AROWS // _NS


def _extract_i32(vec, i):
    return jnp.sum(jnp.where(_iota16() == i, vec, 0))


def _aggregate_sc(xlp, src, dst, lo_flat, mv, invd, qs, heads, e2p):
    dp = heads * 32
    dmsg = 96 if heads > 1 else 32
    bb = 128
    mesh = plsc.VectorSubcoreMesh(core_axis_name="c", subcore_axis_name="s")

    @functools.partial(
        pl.kernel, mesh=mesh, compiler_params=_sc_params(),
        out_type=jax.ShapeDtypeStruct((_NQ, _AROWS, dmsg), jnp.float32),
        scratch_types=[
            pltpu.VMEM((bb,), jnp.int32),
            pltpu.VMEM((bb,), jnp.int32),
            pltpu.VMEM((bb,), jnp.int32),
            pltpu.VMEM((bb,), jnp.int32),
            pltpu.VMEM((bb,), jnp.int32),
            pltpu.VMEM((bb,), jnp.int32),
            pltpu.VMEM((bb, dp), jnp.float32),
            pltpu.VMEM((bb, dp), jnp.float32),
            pltpu.VMEM((bb, dmsg), jnp.float32),
            pltpu.VMEM((bb, dmsg), jnp.float32),
            pltpu.VMEM((heads * bb,), jnp.float32),
            pltpu.VMEM((heads * bb,), jnp.float32),
            pltpu.VMEM((bb, 16), jnp.float32),
            pltpu.VMEM((bb, 16), jnp.float32),
            pltpu.VMEM((16,), jnp.float32),
            pltpu.VMEM((16,), jnp.int32),
            pltpu.VMEM_SHARED((_AROWS, dmsg), jnp.float32),
            pltpu.SemaphoreType.DMA,
            pltpu.SemaphoreType.DMA,
            pltpu.SemaphoreType.DMA,
            pltpu.SemaphoreType.DMA,
            pltpu.SemaphoreType.DMA,
            pltpu.SemaphoreType.DMA,
        ],
    )
    def kern(xl_hbm, src_hbm, dst_hbm, lo_hbm, mv_hbm, inv_hbm, qs_hbm,
             zer_hbm, out_hbm, srcv0, srcv1, dstv0, dstv1, dloc0, dloc1,
             xlr0, xlr1, msg0, msg1, lhv0, lhv1, invr0, invr1, mvv, qsv,
             acc_sp, s0, s1, sg0, sg1, sc0, sc1):
        core = lax.axis_index("c")
        sid = lax.axis_index("s")
        pltpu.sync_copy(mv_hbm, mvv)
        pltpu.sync_copy(qs_hbm, qsv)
        pltpu.sync_copy(zer_hbm, msg0)
        pltpu.sync_copy(zer_hbm, msg1)

        def load(base, srcv, dstv, lhv, sem):
            cps = [pltpu.async_copy(src_hbm.at[pl.ds(base, bb)], srcv, sem),
                   pltpu.async_copy(dst_hbm.at[pl.ds(base, bb)], dstv, sem)]
            for h in range(heads):
                cps.append(pltpu.async_copy(
                    lo_hbm.at[pl.ds(h * e2p + base, bb)],
                    lhv.at[pl.ds(h * bb, bb)], sem))
            return cps

        def compute(base, s_k, e_k, qbase, dstv, dloc, xlr, invr, lhv, msg):
            @plsc.parallel_loop(0, bb // 16, unroll=2)
            def _group(g):
                ev = _iota16() + g * 16
                eg = base + ev
                inq = (eg >= s_k) & (eg < e_k)
                dv = dstv[pl.ds(g * 16, 16)]
                dloc[pl.ds(g * 16, 16)] = jnp.where(inq, dv - qbase, _QN)
                for h in range(heads):
                    l = lhv[pl.ds(h * bb + g * 16, 16)]
                    ex = jnp.exp(l - mvv[...])
                    iv = plsc.load_gather(
                        invr, [ev, jnp.full((16,), h, jnp.int32)])
                    alpha = ex * iv
                    for c in range(_HID):
                        xv = plsc.load_gather(
                            xlr, [ev, jnp.full((16,), h * 32 + c, jnp.int32)])
                        plsc.store_scatter(
                            msg, [ev, jnp.full((16,), h * _HID + c,
                                               jnp.int32)],
                            xv * alpha)

        @pl.loop(0, _NQ // 2)
        def _range(j):
            q = core * (_NQ // 2) + j
            qsvv = qsv[...]
            qlo = _extract_i32(qsvv, q)
            qhi = _extract_i32(qsvv, q + 1)
            qbase = q * _QN
            per_t = (qhi - qlo + _NS - 1) // _NS
            s_k = qlo + sid * per_t
            e_k = jnp.minimum(s_k + per_t, qhi)
            s8 = (s_k // 8) * 8
            nchp = jnp.maximum((e_k - s8 + 2 * bb - 1) // (2 * bb), 0)

            @pl.loop(0, _ASTRIPE // bb)
            def _za(i):
                pltpu.sync_copy(msg0, acc_sp.at[pl.ds(sid * _ASTRIPE + i * bb,
                                                      bb)])

            rem = _ASTRIPE % bb
            if rem:
                pltpu.sync_copy(msg0.at[pl.ds(0, rem)],
                                acc_sp.at[pl.ds(sid * _ASTRIPE
                                                + (_ASTRIPE // bb) * bb, rem)])
            plsc.subcore_barrier()

            def _pair(i, carry):
                base_a = s8 + (2 * i) * bb
                base_b = base_a + bb
                la = load(base_a, srcv0, dstv0, lhv0, s0)
                lb = load(base_b, srcv1, dstv1, lhv1, s1)
                la[0].wait()
                la[1].wait()
                ga = [pltpu.async_copy(xl_hbm.at[srcv0], xlr0, sg0),
                      pltpu.async_copy(inv_hbm.at[dstv0], invr0, sg0)]
                lb[0].wait()
                lb[1].wait()
                gb = [pltpu.async_copy(xl_hbm.at[srcv1], xlr1, sg1),
                      pltpu.async_copy(inv_hbm.at[dstv1], invr1, sg1)]
                for cp in ga + la[2:]:
                    cp.wait()
                compute(base_a, s_k, e_k, qbase, dstv0, dloc0, xlr0, invr0,
                        lhv0, msg0)
                sca = pltpu.async_copy(msg0, acc_sp.at[dloc0], sc0, add=True)
                for cp in gb + lb[2:]:
                    cp.wait()
                compute(base_b, s_k, e_k, qbase, dstv1, dloc1, xlr1, invr1,
                        lhv1, msg1)
                scb = pltpu.async_copy(msg1, acc_sp.at[dloc1], sc1, add=True)
                sca.wait()
                scb.wait()
                return carry

            lax.fori_loop(0, nchp, _pair, 0)
            plsc.subcore_barrier()

            @pl.loop(0, _ASTRIPE // bb)
            def _fl(i):
                pltpu.sync_copy(acc_sp.at[pl.ds(sid * _ASTRIPE + i * bb, bb)],
                                out_hbm.at[q, pl.ds(sid * _ASTRIPE + i * bb,
                                                    bb)])

            if rem:
                pltpu.sync_copy(
                    acc_sp.at[pl.ds(sid * _ASTRIPE + (_ASTRIPE // bb) * bb,
                                    rem)],
                    out_hbm.at[q, pl.ds(sid * _ASTRIPE + (_ASTRIPE // bb) * bb,
                                        rem)])

            pltpu.sync_copy(zer_hbm, msg0)

    return kern(xlp, src, dst, lo_flat, mv, invd, qs,
                jnp.zeros((bb, dmsg), jnp.float32))


_PB = 320
_PROWS = 528
_PSTRIPE = _PROWS // _NS


def _pool_sc(h_pad, batch_pad):
    per_tile = _NP // _NW
    n_chunks = per_tile // _PB
    mesh = plsc.VectorSubcoreMesh(core_axis_name="c", subcore_axis_name="s")

    @functools.partial(
        pl.kernel, mesh=mesh, compiler_params=_sc_params(),
        out_type=[jax.ShapeDtypeStruct((_NC, _PROWS, 32), jnp.float32),
                  jax.ShapeDtypeStruct((_NC, _PROWS, 32), jnp.float32)],
        scratch_types=[
            pltpu.VMEM((_PB,), jnp.int32),
            pltpu.VMEM((_PB, 32), jnp.float32),
            pltpu.VMEM((_PB, 32), jnp.float32),
            pltpu.VMEM_SHARED((_PROWS, 32), jnp.float32),
            pltpu.VMEM_SHARED((_PROWS, 32), jnp.float32),
        ],
    )
    def kern(h_hbm, b_hbm, ones_hbm, zer_hbm, sum_hbm, cnt_hbm, bv, hrows,
             ones, sum_sp, cnt_sp):
        core = lax.axis_index("c")
        sid = lax.axis_index("s")
        wid = sid * _NC + core
        tbase = wid * per_tile
        pltpu.sync_copy(ones_hbm, ones)
        pltpu.sync_copy(zer_hbm, hrows)

        pltpu.sync_copy(hrows.at[pl.ds(0, _PSTRIPE)],
                        sum_sp.at[pl.ds(sid * _PSTRIPE, _PSTRIPE)])
        pltpu.sync_copy(hrows.at[pl.ds(0, _PSTRIPE)],
                        cnt_sp.at[pl.ds(sid * _PSTRIPE, _PSTRIPE)])
        plsc.subcore_barrier()

        @pl.loop(0, n_chunks)
        def _chunk(ci):
            base = tbase + ci * _PB
            pltpu.sync_copy(b_hbm.at[pl.ds(base, _PB)], bv)
            pltpu.sync_copy(h_hbm.at[pl.ds(base, _PB)], hrows)
            pltpu.sync_copy(hrows, sum_sp.at[bv], add=True)
            pltpu.sync_copy(ones, cnt_sp.at[bv], add=True)

        plsc.subcore_barrier()
        pltpu.sync_copy(sum_sp.at[pl.ds(sid * _PSTRIPE, _PSTRIPE)],
                        sum_hbm.at[core, pl.ds(sid * _PSTRIPE, _PSTRIPE)])
        pltpu.sync_copy(cnt_sp.at[pl.ds(sid * _PSTRIPE, _PSTRIPE)],
                        cnt_hbm.at[core, pl.ds(sid * _PSTRIPE, _PSTRIPE)])

    return kern(h_pad, batch_pad, jnp.ones((_PB, 32), jnp.float32),
                jnp.zeros((_PB, 32), jnp.float32))


def _gatv2_layer(h_pad, src, dst, qs, wl, wr, att, bias, heads, e2, e2p):
    wlp = _pad_w(wl, heads)
    wrp = _pad_w(wr, heads)
    xlp, xrp = _proj(h_pad, wlp, wrp)
    att_rep = jnp.repeat(att.reshape(-1), 16)

    lo_flat, mx = _edge_logits_sc(xlp, xrp, src, dst, att_rep, heads, e2p)
    m = jnp.max(mx)
    mv = jnp.full((16,), m, jnp.float32)
    den2 = _den_sc(lo_flat, dst, mv, heads, e2p)
    invd = 1.0 / (den2[0] + den2[1] + 1e-16)
    out4 = _aggregate_sc(xlp, src, dst, lo_flat, mv, invd, qs, heads, e2p)
    d = heads * _HID
    out = out4[:, :_QN, :d].reshape(_NQ * _QN, d)
    return out + bias


def kernel(x, edge_index, batch, Wl1, Wr1, att1, b1, g1, be1, Wl2, Wr2, att2,
           b2, g2, be2, Wl3, Wr3, att3, b3, g3, be3):
    n = 50000
    e = edge_index.shape[1]
    e2 = e + n
    e2p = ((e2 + _NW * _B - 1) // (_NW * _B)) * (_NW * _B)

    loop = jnp.arange(n, dtype=jnp.int32)
    src0 = jnp.concatenate([edge_index[0], loop,
                            jnp.zeros((e2p - e2,), jnp.int32)])
    dst0 = jnp.concatenate([edge_index[1], loop,
                            jnp.full((e2p - e2,), n, jnp.int32)])
    order = jnp.argsort(dst0[:e2])
    src = jnp.concatenate([src0[order], src0[e2:]])
    dst = jnp.concatenate([dst0[order], dst0[e2:]])
    qs = jnp.searchsorted(
        dst, jnp.arange(0, (_NQ + 1) * _QN, _QN, dtype=jnp.int32))
    qs = jnp.concatenate([qs.astype(jnp.int32),
                          jnp.zeros((16 - _NQ - 1,), jnp.int32)])

    def pad_rows(h):
        return jnp.pad(h, ((0, _NP - h.shape[0]), (0, 0)))

    h = pad_rows(x.astype(jnp.float32))
    h = jnp.pad(h, ((0, 0), (0, 2)))
    h = _gatv2_layer(h, src, dst, qs, jnp.pad(Wl1, ((0, 2), (0, 0))),
                     jnp.pad(Wr1, ((0, 2), (0, 0))), att1, b1, _H, e2, e2p)
    h = _bn_elu(pad_rows(h), n, g1, be1)
    h = _gatv2_layer(h, src, dst, qs, Wl2, Wr2, att2, b2, _H, e2, e2p)
    h = _bn_elu(pad_rows(h), n, g2, be2)
    h = _gatv2_layer(h, src, dst, qs, Wl3, Wr3, att3, b3, 1, e2, e2p)
    h = _bn_elu(pad_rows(h), n, g3, be3)

    node_emb = h[:n]
    h32 = jnp.pad(h, ((0, 0), (0, 8)))
    batch_pad = jnp.concatenate([batch.astype(jnp.int32),
                                 jnp.full((_NP - n,), _G, jnp.int32)])
    sums2, cnts2 = _pool_sc(h32, batch_pad)
    sums = (sums2[0] + sums2[1])[:_G, :_HID]
    cnt = (cnts2[0] + cnts2[1])[:_G, :1]
    graph_emb = sums / jnp.maximum(cnt, 1.0)
    return (graph_emb, node_emb)
